# Initial kernel scaffold; baseline (speedup 1.0000x reference)
import jax
import jax.numpy as jnp
from jax.experimental import pallas as pl


def kernel(x_chemical, x_disease, edge_interacts, edge_affects, edge_treated_by, W1_interacts, b1_interacts, W1_affects, b1_affects, W1_treated_by, b1_treated_by, W2_interacts, b2_interacts, W2_affects, b2_affects, W2_treated_by, b2_treated_by):
    raise NotImplementedError("write your pallas kernel here")


# speedup vs baseline: 7.1312x; 7.1312x over previous
import functools

import jax
import jax.numpy as jnp
from jax import lax
from jax.experimental import pallas as pl
from jax.experimental.pallas import tpu as pltpu
from jax.experimental.pallas import tpu_sc as plsc

N = 10000
NP = 10240
D = 128
E = 160000
EP = 163840
NC = 2
NS = 16
L = 128
CH = EP // (NC * NS * L)
RT = NP // NS


def _deg_body(idx_hbm, out_hbm, cidx_v, ones_v, z_v, c0, c1, c2, sem):
    del sem
    c = lax.axis_index("c")
    s = lax.axis_index("s")
    cnts = (c0, c1, c2)
    for i in range(8):
        ones_v[pl.ds(i * 16, 16)] = jnp.full((16,), 1.0, jnp.float32)

    def _zrow(i, _):
        z_v[pl.ds(i * 16, 16)] = jnp.zeros((16,), jnp.float32)
        return 0

    lax.fori_loop(0, RT // 16, _zrow, 0)
    for k in range(3):
        pltpu.sync_copy(z_v, cnts[k].at[pl.ds(s * RT, RT)])
    plsc.subcore_barrier()
    for k in range(3):
        for h in range(NC):
            base = (((3 * c + k) * NC + h) * NS + s) * CH
            pltpu.sync_copy(idx_hbm.at[pl.ds(base, CH)], cidx_v)

            def _chunk(j, _, k=k):
                pltpu.sync_copy(ones_v, cnts[k].at[cidx_v.at[j]], add=True)
                return 0

            lax.fori_loop(0, CH, _chunk, 0)
    plsc.subcore_barrier()
    for k in range(3):
        pltpu.sync_copy(cnts[k].at[pl.ds(s * RT, RT)],
                        out_hbm.at[pl.ds((3 * c + k) * NP + s * RT, RT)])


def _degree_kernel(idx_all):
    mesh = plsc.VectorSubcoreMesh(core_axis_name="c", subcore_axis_name="s")
    return pl.kernel(
        _deg_body,
        out_type=jax.ShapeDtypeStruct((6 * NP,), jnp.float32),
        mesh=mesh,
        scratch_types=[
            pltpu.VMEM((CH, L), jnp.int32),
            pltpu.VMEM((L,), jnp.float32),
            pltpu.VMEM((RT,), jnp.float32),
            pltpu.VMEM_SHARED((NP,), jnp.float32),
            pltpu.VMEM_SHARED((NP,), jnp.float32),
            pltpu.VMEM_SHARED((NP,), jnp.float32),
            pltpu.SemaphoreType.DMA,
        ],
    )(idx_all)


def _agg_body(n_rel, *args):
    srcs = args[0:n_rel]
    dsts = args[n_rel:2 * n_rel]
    tabs = args[2 * n_rel:3 * n_rel]
    outs = args[3 * n_rel:4 * n_rel]
    sidx_v, didx_v, rows_v, zb_v, acc_sh, sem = args[4 * n_rel:]
    c = lax.axis_index("c")
    s = lax.axis_index("s")

    def _zrow(i, _):
        for q in range(8):
            zb_v[i, pl.ds(q * 16, 16)] = jnp.zeros((16,), jnp.float32)
        return 0

    lax.fori_loop(0, L, _zrow, 0)
    for r in range(n_rel):
        for kk in range(RT // L):
            pltpu.sync_copy(zb_v, acc_sh.at[pl.ds(s * RT + kk * L, L)])
        plsc.subcore_barrier()
        pltpu.sync_copy(srcs[r].at[pl.ds((c * NS + s) * CH, CH)], sidx_v)
        pltpu.sync_copy(dsts[r].at[pl.ds((c * NS + s) * CH, CH)], didx_v)

        def _chunk(j, _, r=r):
            pltpu.async_copy(tabs[r].at[sidx_v.at[j]], rows_v, sem).wait()
            pltpu.sync_copy(rows_v, acc_sh.at[didx_v.at[j]], add=True)
            return 0

        lax.fori_loop(0, CH, _chunk, 0)
        plsc.subcore_barrier()
        pltpu.sync_copy(acc_sh.at[pl.ds(s * RT, RT)],
                        outs[r].at[pl.ds(c * NP + s * RT, RT)])
        plsc.subcore_barrier()


def _agg_kernel(srcs, dsts, tabs):
    n_rel = len(srcs)
    mesh = plsc.VectorSubcoreMesh(core_axis_name="c", subcore_axis_name="s")
    out_t = tuple(jax.ShapeDtypeStruct((NC * NP, D), jnp.float32)
                  for _ in range(n_rel))
    return pl.kernel(
        functools.partial(_agg_body, n_rel),
        out_type=out_t,
        mesh=mesh,
        scratch_types=[
            pltpu.VMEM((CH, L), jnp.int32),
            pltpu.VMEM((CH, L), jnp.int32),
            pltpu.VMEM((L, D), jnp.float32),
            pltpu.VMEM((L, D), jnp.float32),
            pltpu.VMEM_SHARED((NP, D), jnp.float32),
            pltpu.SemaphoreType.DMA,
        ],
    )(*srcs, *dsts, *tabs)


def _norm(deg):
    return jnp.where(deg > 0, lax.rsqrt(jnp.maximum(deg, 1e-12)), 0.0)


def _dense1_body(xc, xd, odi, idi, oda, ida, odt, idt, wi, wt, wa,
                 g1i, g1t, g1a, ndi, ndt, nda, nsi, nst):
    nsi_v = _norm(odi[...])
    nst_v = _norm(odt[...])
    nsa_v = _norm(oda[...])
    ndi[...] = _norm(idi[...])
    nda[...] = _norm(ida[...])
    ndt[...] = _norm(idt[...])
    nsi[...] = nsi_v
    nst[...] = nst_v
    xc_v = xc[...]
    xd_v = xd[...]
    g1i[...] = jnp.dot(xc_v * nsi_v, wi[...],
                       preferred_element_type=jnp.float32)
    g1t[...] = jnp.dot(xd_v * nst_v, wt[...],
                       preferred_element_type=jnp.float32)
    g1a[...] = jnp.dot(xc_v * nsa_v, wa[...],
                       preferred_element_type=jnp.float32)


def _dense1(xc, xd, deg, w1i, w1t, w1a):
    grid = NP // RT
    row = pl.BlockSpec((RT, D), lambda i: (i, 0))
    vec = pl.BlockSpec((RT, 1), lambda i: (i, 0))
    wsp = pl.BlockSpec((D, D), lambda i: (0, 0))
    outs = (
        jax.ShapeDtypeStruct((NP, D), jnp.float32),
        jax.ShapeDtypeStruct((NP, D), jnp.float32),
        jax.ShapeDtypeStruct((NP, D), jnp.float32),
        jax.ShapeDtypeStruct((NP, 1), jnp.float32),
        jax.ShapeDtypeStruct((NP, 1), jnp.float32),
        jax.ShapeDtypeStruct((NP, 1), jnp.float32),
        jax.ShapeDtypeStruct((NP, 1), jnp.float32),
        jax.ShapeDtypeStruct((NP, 1), jnp.float32),
    )
    return pl.pallas_call(
        _dense1_body,
        grid=(grid,),
        in_specs=[row, row, vec, vec, vec, vec, vec, vec, wsp, wsp, wsp],
        out_specs=(row, row, row, vec, vec, vec, vec, vec),
        out_shape=outs,
    )(xc, xd, deg[0], deg[1], deg[2], deg[3], deg[4], deg[5], w1i, w1t, w1a)


def _leaky(x):
    return jnp.where(x >= 0, x, 0.01 * x)


def _dense2_body(a1i, a1t, a1a, ndi, ndt, nda, nsi, nst,
                 b1i, b1t, b1a, w2i, w2t, g2i, g2t):
    h_chem = _leaky(ndi[...] * (a1i[0] + a1i[1]) + b1i[...] +
                    ndt[...] * (a1t[0] + a1t[1]) + b1t[...])
    h_dis = _leaky(nda[...] * (a1a[0] + a1a[1]) + b1a[...])
    g2i[...] = jnp.dot(h_chem * nsi[...], w2i[...],
                       preferred_element_type=jnp.float32)
    g2t[...] = jnp.dot(h_dis * nst[...], w2t[...],
                       preferred_element_type=jnp.float32)


def _dense2(a1i, a1t, a1a, ndi, ndt, nda, nsi, nst, b1i, b1t, b1a, w2i, w2t):
    grid = NP // RT
    part = pl.BlockSpec((NC, RT, D), lambda i: (0, i, 0))
    row = pl.BlockSpec((RT, D), lambda i: (i, 0))
    vec = pl.BlockSpec((RT, 1), lambda i: (i, 0))
    bias = pl.BlockSpec((1, D), lambda i: (0, 0))
    wsp = pl.BlockSpec((D, D), lambda i: (0, 0))
    outs = (
        jax.ShapeDtypeStruct((NP, D), jnp.float32),
        jax.ShapeDtypeStruct((NP, D), jnp.float32),
    )
    return pl.pallas_call(
        _dense2_body,
        grid=(grid,),
        in_specs=[part, part, part, vec, vec, vec, vec, vec,
                  bias, bias, bias, wsp, wsp],
        out_specs=(row, row),
        out_shape=outs,
    )(a1i, a1t, a1a, ndi, ndt, nda, nsi, nst, b1i, b1t, b1a, w2i, w2t)


def _dense3_body(a2i, a2t, ndi, ndt, b2i, b2t, out):
    out[...] = (ndi[...] * (a2i[0] + a2i[1]) + b2i[...] +
                ndt[...] * (a2t[0] + a2t[1]) + b2t[...])


def _dense3(a2i, a2t, ndi, ndt, b2i, b2t):
    grid = NP // RT
    part = pl.BlockSpec((NC, RT, D), lambda i: (0, i, 0))
    row = pl.BlockSpec((RT, D), lambda i: (i, 0))
    vec = pl.BlockSpec((RT, 1), lambda i: (i, 0))
    bias = pl.BlockSpec((1, D), lambda i: (0, 0))
    return pl.pallas_call(
        _dense3_body,
        grid=(grid,),
        in_specs=[part, part, vec, vec, bias, bias],
        out_specs=row,
        out_shape=jax.ShapeDtypeStruct((NP, D), jnp.float32),
    )(a2i, a2t, ndi, ndt, b2i, b2t)


def _prep_idx(e):
    pad = N + (jnp.arange(EP - E, dtype=jnp.int32) % (NP - N))
    return jnp.concatenate([e, pad]).reshape(NC * NS * CH, L)


def kernel(x_chemical, x_disease, edge_interacts, edge_affects,
           edge_treated_by,
           W1_interacts, b1_interacts, W1_affects, b1_affects,
           W1_treated_by, b1_treated_by,
           W2_interacts, b2_interacts, W2_affects, b2_affects,
           W2_treated_by, b2_treated_by):
    si = _prep_idx(edge_interacts[0])
    di = _prep_idx(edge_interacts[1])
    sa = _prep_idx(edge_affects[0])
    da = _prep_idx(edge_affects[1])
    st = _prep_idx(edge_treated_by[0])
    dt = _prep_idx(edge_treated_by[1])
    idx_all = jnp.concatenate([si, di, sa, da, st, dt], axis=0)

    xc = jnp.pad(x_chemical, ((0, NP - N), (0, 0)))
    xd = jnp.pad(x_disease, ((0, NP - N), (0, 0)))

    counts = _degree_kernel(idx_all)
    deg = [counts[k * NP:(k + 1) * NP, None] for k in range(6)]

    g1i, g1t, g1a, ndi, ndt, nda, nsi, nst = _dense1(
        xc, xd, deg, W1_interacts, W1_treated_by, W1_affects)

    a1i, a1t, a1a = _agg_kernel((si, st, sa), (di, dt, da), (g1i, g1t, g1a))
    a1i, a1t, a1a = (a.reshape(NC, NP, D) for a in (a1i, a1t, a1a))

    g2i, g2t = _dense2(a1i, a1t, a1a, ndi, ndt, nda, nsi, nst,
                       b1_interacts[None, :], b1_treated_by[None, :],
                       b1_affects[None, :], W2_interacts, W2_treated_by)

    a2i, a2t = _agg_kernel((si, st), (di, dt), (g2i, g2t))
    a2i, a2t = (a.reshape(NC, NP, D) for a in (a2i, a2t))

    out = _dense3(a2i, a2t, ndi, ndt,
                  b2_interacts[None, :], b2_treated_by[None, :])
    return out[:N]


# speedup vs baseline: 9.6500x; 1.3532x over previous
import functools

import jax
import jax.numpy as jnp
from jax import lax
from jax.experimental import pallas as pl
from jax.experimental.pallas import tpu as pltpu
from jax.experimental.pallas import tpu_sc as plsc

N = 10000
NP = 10240
D = 128
E = 160000
EP = 163840
NC = 2
NS = 16
L = 128
CH = EP // (NC * NS * L)
RT = NP // NS


def _deg_body(idx_hbm, out_hbm, cidx_v, ones_v, z_v, c0, c1, c2, sem):
    del sem
    c = lax.axis_index("c")
    s = lax.axis_index("s")
    cnts = (c0, c1, c2)
    for i in range(8):
        ones_v[pl.ds(i * 16, 16)] = jnp.full((16,), 1.0, jnp.float32)

    def _zrow(i, _):
        z_v[pl.ds(i * 16, 16)] = jnp.zeros((16,), jnp.float32)
        return 0

    lax.fori_loop(0, RT // 16, _zrow, 0)
    for k in range(3):
        pltpu.sync_copy(z_v, cnts[k].at[pl.ds(s * RT, RT)])
    plsc.subcore_barrier()
    for k in range(3):
        for h in range(NC):
            base = (((3 * c + k) * NC + h) * NS + s) * CH
            pltpu.sync_copy(idx_hbm.at[pl.ds(base, CH)], cidx_v)

            def _chunk(j, _, k=k):
                pltpu.sync_copy(ones_v, cnts[k].at[cidx_v.at[j]], add=True)
                return 0

            lax.fori_loop(0, CH, _chunk, 0)
    plsc.subcore_barrier()
    for k in range(3):
        pltpu.sync_copy(cnts[k].at[pl.ds(s * RT, RT)],
                        out_hbm.at[pl.ds((3 * c + k) * NP + s * RT, RT)])


def _degree_kernel(idx_all):
    mesh = plsc.VectorSubcoreMesh(core_axis_name="c", subcore_axis_name="s")
    return pl.kernel(
        _deg_body,
        out_type=jax.ShapeDtypeStruct((6 * NP,), jnp.float32),
        mesh=mesh,
        scratch_types=[
            pltpu.VMEM((CH, L), jnp.int32),
            pltpu.VMEM((L,), jnp.float32),
            pltpu.VMEM((RT,), jnp.float32),
            pltpu.VMEM_SHARED((NP,), jnp.float32),
            pltpu.VMEM_SHARED((NP,), jnp.float32),
            pltpu.VMEM_SHARED((NP,), jnp.float32),
            pltpu.SemaphoreType.DMA,
        ],
    )(idx_all)


def _agg_body(n_rel, *args):
    srcs = args[0:n_rel]
    dsts = args[n_rel:2 * n_rel]
    tabs = args[2 * n_rel:3 * n_rel]
    outs = args[3 * n_rel:4 * n_rel]
    sidx_v, didx_v, rows_a, rows_b, acc_sh, sem_a, sem_b = \
        args[4 * n_rel:]
    c = lax.axis_index("c")
    s = lax.axis_index("s")

    def _zrow(i, _):
        for q in range(8):
            rows_a[i, pl.ds(q * 16, 16)] = jnp.zeros((16,), jnp.float32)
        return 0

    for r in range(n_rel):
        lax.fori_loop(0, L, _zrow, 0)
        for kk in range(RT // L):
            pltpu.sync_copy(rows_a, acc_sh.at[pl.ds(s * RT + kk * L, L)])
        plsc.subcore_barrier()
        pltpu.sync_copy(srcs[r].at[pl.ds((c * NS + s) * CH, CH)], sidx_v)
        pltpu.sync_copy(dsts[r].at[pl.ds((c * NS + s) * CH, CH)], didx_v)

        pltpu.async_copy(tabs[r].at[sidx_v.at[0]], rows_a, sem_a)

        def _chunk2(i, _, r=r):
            j = 2 * i
            pltpu.async_copy(tabs[r].at[sidx_v.at[j + 1]], rows_b, sem_b)
            pltpu.make_async_copy(tabs[r].at[sidx_v.at[0]], rows_a,
                                  sem_a).wait()
            pltpu.sync_copy(rows_a, acc_sh.at[didx_v.at[j]], add=True)

            @pl.when(j + 2 < CH)
            def _():
                pltpu.async_copy(tabs[r].at[sidx_v.at[j + 2]], rows_a, sem_a)

            pltpu.make_async_copy(tabs[r].at[sidx_v.at[0]], rows_b,
                                  sem_b).wait()
            pltpu.sync_copy(rows_b, acc_sh.at[didx_v.at[j + 1]], add=True)
            return 0

        lax.fori_loop(0, CH // 2, _chunk2, 0)
        plsc.subcore_barrier()
        pltpu.sync_copy(acc_sh.at[pl.ds(s * RT, RT)],
                        outs[r].at[pl.ds(c * NP + s * RT, RT)])
        plsc.subcore_barrier()


def _agg_kernel(srcs, dsts, tabs):
    n_rel = len(srcs)
    mesh = plsc.VectorSubcoreMesh(core_axis_name="c", subcore_axis_name="s")
    out_t = tuple(jax.ShapeDtypeStruct((NC * NP, D), jnp.float32)
                  for _ in range(n_rel))
    return pl.kernel(
        functools.partial(_agg_body, n_rel),
        out_type=out_t,
        mesh=mesh,
        scratch_types=[
            pltpu.VMEM((CH, L), jnp.int32),
            pltpu.VMEM((CH, L), jnp.int32),
            pltpu.VMEM((L, D), jnp.float32),
            pltpu.VMEM((L, D), jnp.float32),
            pltpu.VMEM_SHARED((NP, D), jnp.float32),
            pltpu.SemaphoreType.DMA,
            pltpu.SemaphoreType.DMA,
        ],
    )(*srcs, *dsts, *tabs)


def _norm(deg):
    return jnp.where(deg > 0, lax.rsqrt(jnp.maximum(deg, 1e-12)), 0.0)


def _dense1_body(xc, xd, odi, idi, oda, ida, odt, idt, wi, wt, wa,
                 g1i, g1t, g1a, ndi, ndt, nda, nsi, nst):
    nsi_v = _norm(odi[...])
    nst_v = _norm(odt[...])
    nsa_v = _norm(oda[...])
    ndi[...] = _norm(idi[...])
    nda[...] = _norm(ida[...])
    ndt[...] = _norm(idt[...])
    nsi[...] = nsi_v
    nst[...] = nst_v
    xc_v = xc[...]
    xd_v = xd[...]
    g1i[...] = jnp.dot(xc_v * nsi_v, wi[...],
                       preferred_element_type=jnp.float32)
    g1t[...] = jnp.dot(xd_v * nst_v, wt[...],
                       preferred_element_type=jnp.float32)
    g1a[...] = jnp.dot(xc_v * nsa_v, wa[...],
                       preferred_element_type=jnp.float32)


def _dense1(xc, xd, deg, w1i, w1t, w1a):
    grid = NP // RT
    row = pl.BlockSpec((RT, D), lambda i: (i, 0))
    vec = pl.BlockSpec((RT, 1), lambda i: (i, 0))
    wsp = pl.BlockSpec((D, D), lambda i: (0, 0))
    outs = (
        jax.ShapeDtypeStruct((NP, D), jnp.float32),
        jax.ShapeDtypeStruct((NP, D), jnp.float32),
        jax.ShapeDtypeStruct((NP, D), jnp.float32),
        jax.ShapeDtypeStruct((NP, 1), jnp.float32),
        jax.ShapeDtypeStruct((NP, 1), jnp.float32),
        jax.ShapeDtypeStruct((NP, 1), jnp.float32),
        jax.ShapeDtypeStruct((NP, 1), jnp.float32),
        jax.ShapeDtypeStruct((NP, 1), jnp.float32),
    )
    return pl.pallas_call(
        _dense1_body,
        grid=(grid,),
        in_specs=[row, row, vec, vec, vec, vec, vec, vec, wsp, wsp, wsp],
        out_specs=(row, row, row, vec, vec, vec, vec, vec),
        out_shape=outs,
    )(xc, xd, deg[0], deg[1], deg[2], deg[3], deg[4], deg[5], w1i, w1t, w1a)


def _leaky(x):
    return jnp.where(x >= 0, x, 0.01 * x)


def _dense2_body(a1i, a1t, a1a, ndi, ndt, nda, nsi, nst,
                 b1i, b1t, b1a, w2i, w2t, g2i, g2t):
    h_chem = _leaky(ndi[...] * (a1i[0] + a1i[1]) + b1i[...] +
                    ndt[...] * (a1t[0] + a1t[1]) + b1t[...])
    h_dis = _leaky(nda[...] * (a1a[0] + a1a[1]) + b1a[...])
    g2i[...] = jnp.dot(h_chem * nsi[...], w2i[...],
                       preferred_element_type=jnp.float32)
    g2t[...] = jnp.dot(h_dis * nst[...], w2t[...],
                       preferred_element_type=jnp.float32)


def _dense2(a1i, a1t, a1a, ndi, ndt, nda, nsi, nst, b1i, b1t, b1a, w2i, w2t):
    grid = NP // RT
    part = pl.BlockSpec((NC, RT, D), lambda i: (0, i, 0))
    row = pl.BlockSpec((RT, D), lambda i: (i, 0))
    vec = pl.BlockSpec((RT, 1), lambda i: (i, 0))
    bias = pl.BlockSpec((1, D), lambda i: (0, 0))
    wsp = pl.BlockSpec((D, D), lambda i: (0, 0))
    outs = (
        jax.ShapeDtypeStruct((NP, D), jnp.float32),
        jax.ShapeDtypeStruct((NP, D), jnp.float32),
    )
    return pl.pallas_call(
        _dense2_body,
        grid=(grid,),
        in_specs=[part, part, part, vec, vec, vec, vec, vec,
                  bias, bias, bias, wsp, wsp],
        out_specs=(row, row),
        out_shape=outs,
    )(a1i, a1t, a1a, ndi, ndt, nda, nsi, nst, b1i, b1t, b1a, w2i, w2t)


def _dense3_body(a2i, a2t, ndi, ndt, b2i, b2t, out):
    out[...] = (ndi[...] * (a2i[0] + a2i[1]) + b2i[...] +
                ndt[...] * (a2t[0] + a2t[1]) + b2t[...])


def _dense3(a2i, a2t, ndi, ndt, b2i, b2t):
    grid = NP // RT
    part = pl.BlockSpec((NC, RT, D), lambda i: (0, i, 0))
    row = pl.BlockSpec((RT, D), lambda i: (i, 0))
    vec = pl.BlockSpec((RT, 1), lambda i: (i, 0))
    bias = pl.BlockSpec((1, D), lambda i: (0, 0))
    return pl.pallas_call(
        _dense3_body,
        grid=(grid,),
        in_specs=[part, part, vec, vec, bias, bias],
        out_specs=row,
        out_shape=jax.ShapeDtypeStruct((NP, D), jnp.float32),
    )(a2i, a2t, ndi, ndt, b2i, b2t)


def _prep_idx(e):
    pad = N + (jnp.arange(EP - E, dtype=jnp.int32) % (NP - N))
    return jnp.concatenate([e, pad]).reshape(NC * NS * CH, L)


def kernel(x_chemical, x_disease, edge_interacts, edge_affects,
           edge_treated_by,
           W1_interacts, b1_interacts, W1_affects, b1_affects,
           W1_treated_by, b1_treated_by,
           W2_interacts, b2_interacts, W2_affects, b2_affects,
           W2_treated_by, b2_treated_by):
    si = _prep_idx(edge_interacts[0])
    di = _prep_idx(edge_interacts[1])
    sa = _prep_idx(edge_affects[0])
    da = _prep_idx(edge_affects[1])
    st = _prep_idx(edge_treated_by[0])
    dt = _prep_idx(edge_treated_by[1])
    idx_all = jnp.concatenate([si, di, sa, da, st, dt], axis=0)

    xc = jnp.pad(x_chemical, ((0, NP - N), (0, 0)))
    xd = jnp.pad(x_disease, ((0, NP - N), (0, 0)))

    counts = _degree_kernel(idx_all)
    deg = [counts[k * NP:(k + 1) * NP, None] for k in range(6)]

    g1i, g1t, g1a, ndi, ndt, nda, nsi, nst = _dense1(
        xc, xd, deg, W1_interacts, W1_treated_by, W1_affects)

    a1i, a1t, a1a = _agg_kernel((si, st, sa), (di, dt, da), (g1i, g1t, g1a))
    a1i, a1t, a1a = (a.reshape(NC, NP, D) for a in (a1i, a1t, a1a))

    g2i, g2t = _dense2(a1i, a1t, a1a, ndi, ndt, nda, nsi, nst,
                       b1_interacts[None, :], b1_treated_by[None, :],
                       b1_affects[None, :], W2_interacts, W2_treated_by)

    a2i, a2t = _agg_kernel((si, st), (di, dt), (g2i, g2t))
    a2i, a2t = (a.reshape(NC, NP, D) for a in (a2i, a2t))

    out = _dense3(a2i, a2t, ndi, ndt,
                  b2_interacts[None, :], b2_treated_by[None, :])
    return out[:N]


# speedup vs baseline: 9.6528x; 1.0003x over previous
import functools

import jax
import jax.numpy as jnp
from jax import lax
from jax.experimental import pallas as pl
from jax.experimental.pallas import tpu as pltpu
from jax.experimental.pallas import tpu_sc as plsc

N = 10000
NP = 10240
D = 128
E = 160000
EP = 163840
NC = 2
NS = 16
L = 128
CH = EP // (NC * NS * L)
RT = NP // NS


def _deg_body(idx_hbm, out_hbm, cidx_v, ones_v, z_v, c0, c1, c2, sem):
    del sem
    c = lax.axis_index("c")
    s = lax.axis_index("s")
    cnts = (c0, c1, c2)
    for i in range(8):
        ones_v[pl.ds(i * 16, 16)] = jnp.full((16,), 1.0, jnp.float32)

    def _zrow(i, _):
        z_v[pl.ds(i * 16, 16)] = jnp.zeros((16,), jnp.float32)
        return 0

    lax.fori_loop(0, RT // 16, _zrow, 0)
    for k in range(3):
        pltpu.sync_copy(z_v, cnts[k].at[pl.ds(s * RT, RT)])
    plsc.subcore_barrier()
    for k in range(3):
        for h in range(NC):
            base = (((3 * c + k) * NC + h) * NS + s) * CH
            pltpu.sync_copy(idx_hbm.at[pl.ds(base, CH)], cidx_v)

            def _chunk(j, _, k=k):
                pltpu.sync_copy(ones_v, cnts[k].at[cidx_v.at[j]], add=True)
                return 0

            lax.fori_loop(0, CH, _chunk, 0)
    plsc.subcore_barrier()
    for k in range(3):
        pltpu.sync_copy(cnts[k].at[pl.ds(s * RT, RT)],
                        out_hbm.at[pl.ds((3 * c + k) * NP + s * RT, RT)])


def _degree_kernel(idx_all):
    mesh = plsc.VectorSubcoreMesh(core_axis_name="c", subcore_axis_name="s")
    return pl.kernel(
        _deg_body,
        out_type=jax.ShapeDtypeStruct((6 * NP,), jnp.float32),
        mesh=mesh,
        scratch_types=[
            pltpu.VMEM((CH, L), jnp.int32),
            pltpu.VMEM((L,), jnp.float32),
            pltpu.VMEM((RT,), jnp.float32),
            pltpu.VMEM_SHARED((NP,), jnp.float32),
            pltpu.VMEM_SHARED((NP,), jnp.float32),
            pltpu.VMEM_SHARED((NP,), jnp.float32),
            pltpu.SemaphoreType.DMA,
        ],
    )(idx_all)


def _agg_body(n_rel, *args):
    srcs = args[0:n_rel]
    dsts = args[n_rel:2 * n_rel]
    tabs = args[2 * n_rel:3 * n_rel]
    outs = args[3 * n_rel:4 * n_rel]
    sidx_v, didx_v, rows_a, rows_b, acc_sh, sem_a, sem_b = \
        args[4 * n_rel:]
    c = lax.axis_index("c")
    s = lax.axis_index("s")

    def _zrow(i, _):
        for q in range(8):
            rows_a[i, pl.ds(q * 16, 16)] = jnp.zeros((16,), jnp.float32)
        return 0

    for r in range(n_rel):
        lax.fori_loop(0, L, _zrow, 0)
        for kk in range(RT // L):
            pltpu.sync_copy(rows_a, acc_sh.at[pl.ds(s * RT + kk * L, L)])
        plsc.subcore_barrier()
        pltpu.sync_copy(srcs[r].at[pl.ds((c * NS + s) * CH, CH)], sidx_v)
        pltpu.sync_copy(dsts[r].at[pl.ds((c * NS + s) * CH, CH)], didx_v)

        pltpu.async_copy(tabs[r].at[sidx_v.at[0]], rows_a, sem_a)

        def _chunk2(i, _, r=r):
            j = 2 * i
            pltpu.async_copy(tabs[r].at[sidx_v.at[j + 1]], rows_b, sem_b)
            pltpu.make_async_copy(tabs[r].at[sidx_v.at[0]], rows_a,
                                  sem_a).wait()
            pltpu.sync_copy(rows_a, acc_sh.at[didx_v.at[j]], add=True)

            @pl.when(j + 2 < CH)
            def _():
                pltpu.async_copy(tabs[r].at[sidx_v.at[j + 2]], rows_a, sem_a)

            pltpu.make_async_copy(tabs[r].at[sidx_v.at[0]], rows_b,
                                  sem_b).wait()
            pltpu.sync_copy(rows_b, acc_sh.at[didx_v.at[j + 1]], add=True)
            return 0

        lax.fori_loop(0, CH // 2, _chunk2, 0)
        plsc.subcore_barrier()
        pltpu.sync_copy(acc_sh.at[pl.ds(s * RT, RT)],
                        outs[r].at[pl.ds(c * NP + s * RT, RT)])


def _agg_kernel(srcs, dsts, tabs):
    n_rel = len(srcs)
    mesh = plsc.VectorSubcoreMesh(core_axis_name="c", subcore_axis_name="s")
    out_t = tuple(jax.ShapeDtypeStruct((NC * NP, D), jnp.float32)
                  for _ in range(n_rel))
    return pl.kernel(
        functools.partial(_agg_body, n_rel),
        out_type=out_t,
        mesh=mesh,
        scratch_types=[
            pltpu.VMEM((CH, L), jnp.int32),
            pltpu.VMEM((CH, L), jnp.int32),
            pltpu.VMEM((L, D), jnp.float32),
            pltpu.VMEM((L, D), jnp.float32),
            pltpu.VMEM_SHARED((NP, D), jnp.float32),
            pltpu.SemaphoreType.DMA,
            pltpu.SemaphoreType.DMA,
        ],
    )(*srcs, *dsts, *tabs)


def _norm(deg):
    return jnp.where(deg > 0, lax.rsqrt(jnp.maximum(deg, 1e-12)), 0.0)


def _dense0_body(xc, xd, wi, wt, wa, y1i, y1t, y1a):
    xc_v = xc[...]
    y1i[...] = jnp.dot(xc_v, wi[...], preferred_element_type=jnp.float32)
    y1t[...] = jnp.dot(xd[...], wt[...], preferred_element_type=jnp.float32)
    y1a[...] = jnp.dot(xc_v, wa[...], preferred_element_type=jnp.float32)


def _dense0(xc, xd, w1i, w1t, w1a):
    grid = NP // RT
    row = pl.BlockSpec((RT, D), lambda i: (i, 0))
    wsp = pl.BlockSpec((D, D), lambda i: (0, 0))
    outs = tuple(jax.ShapeDtypeStruct((NP, D), jnp.float32)
                 for _ in range(3))
    return pl.pallas_call(
        _dense0_body,
        grid=(grid,),
        in_specs=[row, row, wsp, wsp, wsp],
        out_specs=(row, row, row),
        out_shape=outs,
    )(xc, xd, w1i, w1t, w1a)


def _dense1_body(y1i, y1t, y1a, odi, idi, oda, ida, odt, idt,
                 g1i, g1t, g1a, ndi, ndt, nda, nsi, nst):
    nsi_v = _norm(odi[...])
    nst_v = _norm(odt[...])
    nsa_v = _norm(oda[...])
    ndi[...] = _norm(idi[...])
    nda[...] = _norm(ida[...])
    ndt[...] = _norm(idt[...])
    nsi[...] = nsi_v
    nst[...] = nst_v
    g1i[...] = y1i[...] * nsi_v
    g1t[...] = y1t[...] * nst_v
    g1a[...] = y1a[...] * nsa_v


def _dense1(y1i, y1t, y1a, deg):
    grid = NP // RT
    row = pl.BlockSpec((RT, D), lambda i: (i, 0))
    vec = pl.BlockSpec((RT, 1), lambda i: (i, 0))
    outs = (
        jax.ShapeDtypeStruct((NP, D), jnp.float32),
        jax.ShapeDtypeStruct((NP, D), jnp.float32),
        jax.ShapeDtypeStruct((NP, D), jnp.float32),
        jax.ShapeDtypeStruct((NP, 1), jnp.float32),
        jax.ShapeDtypeStruct((NP, 1), jnp.float32),
        jax.ShapeDtypeStruct((NP, 1), jnp.float32),
        jax.ShapeDtypeStruct((NP, 1), jnp.float32),
        jax.ShapeDtypeStruct((NP, 1), jnp.float32),
    )
    return pl.pallas_call(
        _dense1_body,
        grid=(grid,),
        in_specs=[row, row, row, vec, vec, vec, vec, vec, vec],
        out_specs=(row, row, row, vec, vec, vec, vec, vec),
        out_shape=outs,
    )(y1i, y1t, y1a, deg[0], deg[1], deg[2], deg[3], deg[4], deg[5])


def _leaky(x):
    return jnp.where(x >= 0, x, 0.01 * x)


def _dense2_body(a1i, a1t, a1a, ndi, ndt, nda, nsi, nst,
                 b1i, b1t, b1a, w2i, w2t, g2i, g2t):
    h_chem = _leaky(ndi[...] * (a1i[0] + a1i[1]) + b1i[...] +
                    ndt[...] * (a1t[0] + a1t[1]) + b1t[...])
    h_dis = _leaky(nda[...] * (a1a[0] + a1a[1]) + b1a[...])
    g2i[...] = jnp.dot(h_chem * nsi[...], w2i[...],
                       preferred_element_type=jnp.float32)
    g2t[...] = jnp.dot(h_dis * nst[...], w2t[...],
                       preferred_element_type=jnp.float32)


def _dense2(a1i, a1t, a1a, ndi, ndt, nda, nsi, nst, b1i, b1t, b1a, w2i, w2t):
    grid = NP // RT
    part = pl.BlockSpec((NC, RT, D), lambda i: (0, i, 0))
    row = pl.BlockSpec((RT, D), lambda i: (i, 0))
    vec = pl.BlockSpec((RT, 1), lambda i: (i, 0))
    bias = pl.BlockSpec((1, D), lambda i: (0, 0))
    wsp = pl.BlockSpec((D, D), lambda i: (0, 0))
    outs = (
        jax.ShapeDtypeStruct((NP, D), jnp.float32),
        jax.ShapeDtypeStruct((NP, D), jnp.float32),
    )
    return pl.pallas_call(
        _dense2_body,
        grid=(grid,),
        in_specs=[part, part, part, vec, vec, vec, vec, vec,
                  bias, bias, bias, wsp, wsp],
        out_specs=(row, row),
        out_shape=outs,
    )(a1i, a1t, a1a, ndi, ndt, nda, nsi, nst, b1i, b1t, b1a, w2i, w2t)


def _dense3_body(a2i, a2t, ndi, ndt, b2i, b2t, out):
    out[...] = (ndi[...] * (a2i[0] + a2i[1]) + b2i[...] +
                ndt[...] * (a2t[0] + a2t[1]) + b2t[...])


def _dense3(a2i, a2t, ndi, ndt, b2i, b2t):
    grid = NP // RT
    part = pl.BlockSpec((NC, RT, D), lambda i: (0, i, 0))
    row = pl.BlockSpec((RT, D), lambda i: (i, 0))
    vec = pl.BlockSpec((RT, 1), lambda i: (i, 0))
    bias = pl.BlockSpec((1, D), lambda i: (0, 0))
    return pl.pallas_call(
        _dense3_body,
        grid=(grid,),
        in_specs=[part, part, vec, vec, bias, bias],
        out_specs=row,
        out_shape=jax.ShapeDtypeStruct((NP, D), jnp.float32),
    )(a2i, a2t, ndi, ndt, b2i, b2t)


def _prep_idx(e):
    pad = N + (jnp.arange(EP - E, dtype=jnp.int32) % (NP - N))
    return jnp.concatenate([e, pad]).reshape(NC * NS * CH, L)


def kernel(x_chemical, x_disease, edge_interacts, edge_affects,
           edge_treated_by,
           W1_interacts, b1_interacts, W1_affects, b1_affects,
           W1_treated_by, b1_treated_by,
           W2_interacts, b2_interacts, W2_affects, b2_affects,
           W2_treated_by, b2_treated_by):
    si = _prep_idx(edge_interacts[0])
    di = _prep_idx(edge_interacts[1])
    sa = _prep_idx(edge_affects[0])
    da = _prep_idx(edge_affects[1])
    st = _prep_idx(edge_treated_by[0])
    dt = _prep_idx(edge_treated_by[1])
    idx_all = jnp.concatenate([si, di, sa, da, st, dt], axis=0)

    xc = jnp.pad(x_chemical, ((0, NP - N), (0, 0)))
    xd = jnp.pad(x_disease, ((0, NP - N), (0, 0)))

    counts = _degree_kernel(idx_all)
    deg = [counts[k * NP:(k + 1) * NP, None] for k in range(6)]

    y1i, y1t, y1a = _dense0(xc, xd, W1_interacts, W1_treated_by, W1_affects)
    g1i, g1t, g1a, ndi, ndt, nda, nsi, nst = _dense1(y1i, y1t, y1a, deg)

    a1i, a1t, a1a = _agg_kernel((si, st, sa), (di, dt, da), (g1i, g1t, g1a))
    a1i, a1t, a1a = (a.reshape(NC, NP, D) for a in (a1i, a1t, a1a))

    g2i, g2t = _dense2(a1i, a1t, a1a, ndi, ndt, nda, nsi, nst,
                       b1_interacts[None, :], b1_treated_by[None, :],
                       b1_affects[None, :], W2_interacts, W2_treated_by)

    a2i, a2t = _agg_kernel((si, st), (di, dt), (g2i, g2t))
    a2i, a2t = (a.reshape(NC, NP, D) for a in (a2i, a2t))

    out = _dense3(a2i, a2t, ndi, ndt,
                  b2_interacts[None, :], b2_treated_by[None, :])
    return out[:N]


# speedup vs baseline: 9.8914x; 1.0247x over previous
import functools

import jax
import jax.numpy as jnp
from jax import lax
from jax.experimental import pallas as pl
from jax.experimental.pallas import tpu as pltpu
from jax.experimental.pallas import tpu_sc as plsc

N = 10000
NP = 10240
D = 128
E = 160000
EP = 163840
NC = 2
NS = 16
L = 128
CH = EP // (NC * NS * L)
RT = NP // NS


def _deg_body(a0, a1, a2, a3, a4, a5, out_hbm,
              cidx_v, ones_v, z_v, c0, c1, c2, sem):
    del sem
    c = lax.axis_index("c")
    s = lax.axis_index("s")
    cnts = (c0, c1, c2)
    for i in range(8):
        ones_v[pl.ds(i * 16, 16)] = jnp.full((16,), 1.0, jnp.float32)

    def _zrow(i, _):
        z_v[pl.ds(i * 16, 16)] = jnp.zeros((16,), jnp.float32)
        return 0

    lax.fori_loop(0, RT // 16, _zrow, 0)
    for k in range(3):
        pltpu.sync_copy(z_v, cnts[k].at[pl.ds(s * RT, RT)])
    plsc.subcore_barrier()
    for cc, arrs in ((0, (a0, a1, a2)), (1, (a3, a4, a5))):
        @pl.when(c == cc)
        def _():
            for k in range(3):
                for h in range(NC):
                    base = (h * NS + s) * CH
                    pltpu.sync_copy(arrs[k].at[pl.ds(base, CH)], cidx_v)

                    def _chunk(j, _, k=k):
                        pltpu.sync_copy(ones_v, cnts[k].at[cidx_v.at[j]],
                                        add=True)
                        return 0

                    lax.fori_loop(0, CH, _chunk, 0)
    plsc.subcore_barrier()
    for k in range(3):
        pltpu.sync_copy(cnts[k].at[pl.ds(s * RT, RT)],
                        out_hbm.at[pl.ds((3 * c + k) * NP + s * RT, RT)])


def _degree_kernel(arrs):
    mesh = plsc.VectorSubcoreMesh(core_axis_name="c", subcore_axis_name="s")
    return pl.kernel(
        _deg_body,
        out_type=jax.ShapeDtypeStruct((6 * NP,), jnp.float32),
        mesh=mesh,
        scratch_types=[
            pltpu.VMEM((CH, L), jnp.int32),
            pltpu.VMEM((L,), jnp.float32),
            pltpu.VMEM((RT,), jnp.float32),
            pltpu.VMEM_SHARED((NP,), jnp.float32),
            pltpu.VMEM_SHARED((NP,), jnp.float32),
            pltpu.VMEM_SHARED((NP,), jnp.float32),
            pltpu.SemaphoreType.DMA,
        ],
    )(*arrs)


def _agg_body(n_rel, *args):
    srcs = args[0:n_rel]
    dsts = args[n_rel:2 * n_rel]
    tabs = args[2 * n_rel:3 * n_rel]
    outs = args[3 * n_rel:4 * n_rel]
    sidx_v, didx_v, rows_a, rows_b, acc_sh, sem_a, sem_b = \
        args[4 * n_rel:]
    c = lax.axis_index("c")
    s = lax.axis_index("s")

    def _zrow(i, _):
        for q in range(8):
            rows_a[i, pl.ds(q * 16, 16)] = jnp.zeros((16,), jnp.float32)
        return 0

    for r in range(n_rel):
        lax.fori_loop(0, L, _zrow, 0)
        for kk in range(RT // L):
            pltpu.sync_copy(rows_a, acc_sh.at[pl.ds(s * RT + kk * L, L)])
        plsc.subcore_barrier()
        pltpu.sync_copy(srcs[r].at[pl.ds((c * NS + s) * CH, CH)], sidx_v)
        pltpu.sync_copy(dsts[r].at[pl.ds((c * NS + s) * CH, CH)], didx_v)

        pltpu.async_copy(tabs[r].at[sidx_v.at[0]], rows_a, sem_a)

        def _chunk2(i, _, r=r):
            j = 2 * i
            pltpu.async_copy(tabs[r].at[sidx_v.at[j + 1]], rows_b, sem_b)
            pltpu.make_async_copy(tabs[r].at[sidx_v.at[0]], rows_a,
                                  sem_a).wait()
            pltpu.sync_copy(rows_a, acc_sh.at[didx_v.at[j]], add=True)

            @pl.when(j + 2 < CH)
            def _():
                pltpu.async_copy(tabs[r].at[sidx_v.at[j + 2]], rows_a, sem_a)

            pltpu.make_async_copy(tabs[r].at[sidx_v.at[0]], rows_b,
                                  sem_b).wait()
            pltpu.sync_copy(rows_b, acc_sh.at[didx_v.at[j + 1]], add=True)
            return 0

        lax.fori_loop(0, CH // 2, _chunk2, 0)
        plsc.subcore_barrier()
        pltpu.sync_copy(acc_sh.at[pl.ds(s * RT, RT)],
                        outs[r].at[pl.ds(c * NP + s * RT, RT)])


def _agg_kernel(srcs, dsts, tabs):
    n_rel = len(srcs)
    mesh = plsc.VectorSubcoreMesh(core_axis_name="c", subcore_axis_name="s")
    out_t = tuple(jax.ShapeDtypeStruct((NC * NP, D), jnp.float32)
                  for _ in range(n_rel))
    return pl.kernel(
        functools.partial(_agg_body, n_rel),
        out_type=out_t,
        mesh=mesh,
        scratch_types=[
            pltpu.VMEM((CH, L), jnp.int32),
            pltpu.VMEM((CH, L), jnp.int32),
            pltpu.VMEM((L, D), jnp.float32),
            pltpu.VMEM((L, D), jnp.float32),
            pltpu.VMEM_SHARED((NP, D), jnp.float32),
            pltpu.SemaphoreType.DMA,
            pltpu.SemaphoreType.DMA,
        ],
    )(*srcs, *dsts, *tabs)


def _norm(deg):
    return jnp.where(deg > 0, lax.rsqrt(jnp.maximum(deg, 1e-12)), 0.0)


def _dense1_body(xc, xd, odi, idi, oda, ida, odt, idt, wi, wt, wa,
                 g1i, g1t, g1a, ndi, ndt, nda, nsi, nst):
    nsi_v = _norm(odi[...])
    nst_v = _norm(odt[...])
    nsa_v = _norm(oda[...])
    ndi[...] = _norm(idi[...])
    nda[...] = _norm(ida[...])
    ndt[...] = _norm(idt[...])
    nsi[...] = nsi_v
    nst[...] = nst_v
    xc_v = xc[...]
    xd_v = xd[...]
    g1i[...] = jnp.dot(xc_v * nsi_v, wi[...],
                       preferred_element_type=jnp.float32)
    g1t[...] = jnp.dot(xd_v * nst_v, wt[...],
                       preferred_element_type=jnp.float32)
    g1a[...] = jnp.dot(xc_v * nsa_v, wa[...],
                       preferred_element_type=jnp.float32)


def _dense1(xc, xd, deg, w1i, w1t, w1a):
    grid = NP // RT
    row = pl.BlockSpec((RT, D), lambda i: (i, 0))
    vec = pl.BlockSpec((RT, 1), lambda i: (i, 0))
    wsp = pl.BlockSpec((D, D), lambda i: (0, 0))
    outs = (
        jax.ShapeDtypeStruct((NP, D), jnp.float32),
        jax.ShapeDtypeStruct((NP, D), jnp.float32),
        jax.ShapeDtypeStruct((NP, D), jnp.float32),
        jax.ShapeDtypeStruct((NP, 1), jnp.float32),
        jax.ShapeDtypeStruct((NP, 1), jnp.float32),
        jax.ShapeDtypeStruct((NP, 1), jnp.float32),
        jax.ShapeDtypeStruct((NP, 1), jnp.float32),
        jax.ShapeDtypeStruct((NP, 1), jnp.float32),
    )
    return pl.pallas_call(
        _dense1_body,
        grid=(grid,),
        in_specs=[row, row, vec, vec, vec, vec, vec, vec, wsp, wsp, wsp],
        out_specs=(row, row, row, vec, vec, vec, vec, vec),
        out_shape=outs,
    )(xc, xd, deg[0], deg[1], deg[2], deg[3], deg[4], deg[5], w1i, w1t, w1a)


def _leaky(x):
    return jnp.where(x >= 0, x, 0.01 * x)


def _dense2_body(a1i, a1t, a1a, ndi, ndt, nda, nsi, nst,
                 b1i, b1t, b1a, w2i, w2t, g2i, g2t):
    h_chem = _leaky(ndi[...] * (a1i[0] + a1i[1]) + b1i[...] +
                    ndt[...] * (a1t[0] + a1t[1]) + b1t[...])
    h_dis = _leaky(nda[...] * (a1a[0] + a1a[1]) + b1a[...])
    g2i[...] = jnp.dot(h_chem * nsi[...], w2i[...],
                       preferred_element_type=jnp.float32)
    g2t[...] = jnp.dot(h_dis * nst[...], w2t[...],
                       preferred_element_type=jnp.float32)


def _dense2(a1i, a1t, a1a, ndi, ndt, nda, nsi, nst, b1i, b1t, b1a, w2i, w2t):
    grid = NP // RT
    part = pl.BlockSpec((NC, RT, D), lambda i: (0, i, 0))
    row = pl.BlockSpec((RT, D), lambda i: (i, 0))
    vec = pl.BlockSpec((RT, 1), lambda i: (i, 0))
    bias = pl.BlockSpec((1, D), lambda i: (0, 0))
    wsp = pl.BlockSpec((D, D), lambda i: (0, 0))
    outs = (
        jax.ShapeDtypeStruct((NP, D), jnp.float32),
        jax.ShapeDtypeStruct((NP, D), jnp.float32),
    )
    return pl.pallas_call(
        _dense2_body,
        grid=(grid,),
        in_specs=[part, part, part, vec, vec, vec, vec, vec,
                  bias, bias, bias, wsp, wsp],
        out_specs=(row, row),
        out_shape=outs,
    )(a1i, a1t, a1a, ndi, ndt, nda, nsi, nst, b1i, b1t, b1a, w2i, w2t)


def _dense3_body(a2i, a2t, ndi, ndt, b2i, b2t, out):
    out[...] = (ndi[...] * (a2i[0] + a2i[1]) + b2i[...] +
                ndt[...] * (a2t[0] + a2t[1]) + b2t[...])


def _dense3(a2i, a2t, ndi, ndt, b2i, b2t):
    grid = NP // RT
    part = pl.BlockSpec((NC, RT, D), lambda i: (0, i, 0))
    row = pl.BlockSpec((RT, D), lambda i: (i, 0))
    vec = pl.BlockSpec((RT, 1), lambda i: (i, 0))
    bias = pl.BlockSpec((1, D), lambda i: (0, 0))
    return pl.pallas_call(
        _dense3_body,
        grid=(grid,),
        in_specs=[part, part, vec, vec, bias, bias],
        out_specs=row,
        out_shape=jax.ShapeDtypeStruct((NP, D), jnp.float32),
    )(a2i, a2t, ndi, ndt, b2i, b2t)


def _prep_idx(e):
    pad = N + (jnp.arange(EP - E, dtype=jnp.int32) % (NP - N))
    return jnp.concatenate([e, pad]).reshape(NC * NS * CH, L)


def kernel(x_chemical, x_disease, edge_interacts, edge_affects,
           edge_treated_by,
           W1_interacts, b1_interacts, W1_affects, b1_affects,
           W1_treated_by, b1_treated_by,
           W2_interacts, b2_interacts, W2_affects, b2_affects,
           W2_treated_by, b2_treated_by):
    si = _prep_idx(edge_interacts[0])
    di = _prep_idx(edge_interacts[1])
    sa = _prep_idx(edge_affects[0])
    da = _prep_idx(edge_affects[1])
    st = _prep_idx(edge_treated_by[0])
    dt = _prep_idx(edge_treated_by[1])
    xc = jnp.pad(x_chemical, ((0, NP - N), (0, 0)))
    xd = jnp.pad(x_disease, ((0, NP - N), (0, 0)))

    counts = _degree_kernel((si, di, sa, da, st, dt))
    deg = [counts[k * NP:(k + 1) * NP, None] for k in range(6)]

    g1i, g1t, g1a, ndi, ndt, nda, nsi, nst = _dense1(
        xc, xd, deg, W1_interacts, W1_treated_by, W1_affects)

    a1i, a1t, a1a = _agg_kernel((si, st, sa), (di, dt, da), (g1i, g1t, g1a))
    a1i, a1t, a1a = (a.reshape(NC, NP, D) for a in (a1i, a1t, a1a))

    g2i, g2t = _dense2(a1i, a1t, a1a, ndi, ndt, nda, nsi, nst,
                       b1_interacts[None, :], b1_treated_by[None, :],
                       b1_affects[None, :], W2_interacts, W2_treated_by)

    a2i, a2t = _agg_kernel((si, st), (di, dt), (g2i, g2t))
    a2i, a2t = (a.reshape(NC, NP, D) for a in (a2i, a2t))

    out = _dense3(a2i, a2t, ndi, ndt,
                  b2_interacts[None, :], b2_treated_by[None, :])
    return out[:N]


# speedup vs baseline: 10.1848x; 1.0297x over previous
import functools

import jax
import jax.numpy as jnp
from jax import lax
from jax.experimental import pallas as pl
from jax.experimental.pallas import tpu as pltpu
from jax.experimental.pallas import tpu_sc as plsc

N = 10000
NP = 10240
D = 128
E = 160000
EP = 163840
NC = 2
NS = 16
L = 128
CH = EP // (NC * NS * L)
RT = NP // NS


def _deg_body(a0, a1, a2, a3, a4, a5, out_hbm,
              cidx_v, ones_v, z_v, c0, c1, c2, sem):
    c = lax.axis_index("c")
    s = lax.axis_index("s")
    cnts = (c0, c1, c2)
    for i in range(8):
        ones_v[pl.ds(i * 16, 16)] = jnp.full((16,), 1.0, jnp.float32)

    def _zrow(i, _):
        z_v[pl.ds(i * 16, 16)] = jnp.zeros((16,), jnp.float32)
        return 0

    lax.fori_loop(0, RT // 16, _zrow, 0)
    for k in range(3):
        pltpu.sync_copy(z_v, cnts[k].at[pl.ds(s * RT, RT)])
    plsc.subcore_barrier()
    for cc, arrs in ((0, (a0, a1, a2)), (1, (a3, a4, a5))):
        @pl.when(c == cc)
        def _():
            for k in range(3):
                for h in range(NC):
                    base = (h * NS + s) * CH
                    pltpu.sync_copy(arrs[k].at[pl.ds(base, CH)], cidx_v)

                    def _batch(b, _, k=k):
                        for q in range(8):
                            pltpu.async_copy(
                                ones_v, cnts[k].at[cidx_v.at[8 * b + q]],
                                sem, add=True)
                        for q in range(8):
                            pltpu.make_async_copy(
                                ones_v, cnts[k].at[cidx_v.at[8 * b + q]],
                                sem).wait()
                        return 0

                    lax.fori_loop(0, CH // 8, _batch, 0)
    plsc.subcore_barrier()
    for k in range(3):
        pltpu.sync_copy(cnts[k].at[pl.ds(s * RT, RT)],
                        out_hbm.at[pl.ds((3 * c + k) * NP + s * RT, RT)])


def _degree_kernel(arrs):
    mesh = plsc.VectorSubcoreMesh(core_axis_name="c", subcore_axis_name="s")
    return pl.kernel(
        _deg_body,
        out_type=jax.ShapeDtypeStruct((6 * NP,), jnp.float32),
        mesh=mesh,
        scratch_types=[
            pltpu.VMEM((CH, L), jnp.int32),
            pltpu.VMEM((L,), jnp.float32),
            pltpu.VMEM((RT,), jnp.float32),
            pltpu.VMEM_SHARED((NP,), jnp.float32),
            pltpu.VMEM_SHARED((NP,), jnp.float32),
            pltpu.VMEM_SHARED((NP,), jnp.float32),
            pltpu.SemaphoreType.DMA,
        ],
    )(*arrs)


def _agg_body(n_rel, *args):
    srcs = args[0:n_rel]
    dsts = args[n_rel:2 * n_rel]
    tabs = args[2 * n_rel:3 * n_rel]
    outs = args[3 * n_rel:4 * n_rel]
    sidx_v, didx_v, rows_a, rows_b, acc_sh, sem_a, sem_b = \
        args[4 * n_rel:]
    c = lax.axis_index("c")
    s = lax.axis_index("s")

    def _zrow(i, _):
        for q in range(8):
            rows_a[i, pl.ds(q * 16, 16)] = jnp.zeros((16,), jnp.float32)
        return 0

    for r in range(n_rel):
        lax.fori_loop(0, L, _zrow, 0)
        for kk in range(RT // L):
            pltpu.async_copy(rows_a, acc_sh.at[pl.ds(s * RT + kk * L, L)],
                             sem_a)
        for kk in range(RT // L):
            pltpu.make_async_copy(rows_a,
                                  acc_sh.at[pl.ds(s * RT + kk * L, L)],
                                  sem_a).wait()
        plsc.subcore_barrier()
        pltpu.sync_copy(srcs[r].at[pl.ds((c * NS + s) * CH, CH)], sidx_v)
        pltpu.sync_copy(dsts[r].at[pl.ds((c * NS + s) * CH, CH)], didx_v)

        pltpu.async_copy(tabs[r].at[sidx_v.at[0]], rows_a, sem_a)

        def _chunk2(i, _, r=r):
            j = 2 * i
            pltpu.async_copy(tabs[r].at[sidx_v.at[j + 1]], rows_b, sem_b)
            pltpu.make_async_copy(tabs[r].at[sidx_v.at[0]], rows_a,
                                  sem_a).wait()
            pltpu.sync_copy(rows_a, acc_sh.at[didx_v.at[j]], add=True)

            @pl.when(j + 2 < CH)
            def _():
                pltpu.async_copy(tabs[r].at[sidx_v.at[j + 2]], rows_a, sem_a)

            pltpu.make_async_copy(tabs[r].at[sidx_v.at[0]], rows_b,
                                  sem_b).wait()
            pltpu.sync_copy(rows_b, acc_sh.at[didx_v.at[j + 1]], add=True)
            return 0

        lax.fori_loop(0, CH // 2, _chunk2, 0)
        plsc.subcore_barrier()
        pltpu.sync_copy(acc_sh.at[pl.ds(s * RT, RT)],
                        outs[r].at[pl.ds(c * NP + s * RT, RT)])


def _agg_kernel(srcs, dsts, tabs):
    n_rel = len(srcs)
    mesh = plsc.VectorSubcoreMesh(core_axis_name="c", subcore_axis_name="s")
    out_t = tuple(jax.ShapeDtypeStruct((NC * NP, D), jnp.float32)
                  for _ in range(n_rel))
    return pl.kernel(
        functools.partial(_agg_body, n_rel),
        out_type=out_t,
        mesh=mesh,
        scratch_types=[
            pltpu.VMEM((CH, L), jnp.int32),
            pltpu.VMEM((CH, L), jnp.int32),
            pltpu.VMEM((L, D), jnp.float32),
            pltpu.VMEM((L, D), jnp.float32),
            pltpu.VMEM_SHARED((NP, D), jnp.float32),
            pltpu.SemaphoreType.DMA,
            pltpu.SemaphoreType.DMA,
        ],
    )(*srcs, *dsts, *tabs)


def _norm(deg):
    return jnp.where(deg > 0, lax.rsqrt(jnp.maximum(deg, 1e-12)), 0.0)


def _dense1_body(xc, xd, odi, idi, oda, ida, odt, idt, wi, wt, wa,
                 g1i, g1t, g1a, ndi, ndt, nda, nsi, nst):
    nsi_v = _norm(odi[...])
    nst_v = _norm(odt[...])
    nsa_v = _norm(oda[...])
    ndi[...] = _norm(idi[...])
    nda[...] = _norm(ida[...])
    ndt[...] = _norm(idt[...])
    nsi[...] = nsi_v
    nst[...] = nst_v
    xc_v = xc[...]
    xd_v = xd[...]
    g1i[...] = jnp.dot(xc_v * nsi_v, wi[...],
                       preferred_element_type=jnp.float32)
    g1t[...] = jnp.dot(xd_v * nst_v, wt[...],
                       preferred_element_type=jnp.float32)
    g1a[...] = jnp.dot(xc_v * nsa_v, wa[...],
                       preferred_element_type=jnp.float32)


def _dense1(xc, xd, deg, w1i, w1t, w1a):
    grid = NP // RT
    row = pl.BlockSpec((RT, D), lambda i: (i, 0))
    vec = pl.BlockSpec((RT, 1), lambda i: (i, 0))
    wsp = pl.BlockSpec((D, D), lambda i: (0, 0))
    outs = (
        jax.ShapeDtypeStruct((NP, D), jnp.float32),
        jax.ShapeDtypeStruct((NP, D), jnp.float32),
        jax.ShapeDtypeStruct((NP, D), jnp.float32),
        jax.ShapeDtypeStruct((NP, 1), jnp.float32),
        jax.ShapeDtypeStruct((NP, 1), jnp.float32),
        jax.ShapeDtypeStruct((NP, 1), jnp.float32),
        jax.ShapeDtypeStruct((NP, 1), jnp.float32),
        jax.ShapeDtypeStruct((NP, 1), jnp.float32),
    )
    return pl.pallas_call(
        _dense1_body,
        grid=(grid,),
        in_specs=[row, row, vec, vec, vec, vec, vec, vec, wsp, wsp, wsp],
        out_specs=(row, row, row, vec, vec, vec, vec, vec),
        out_shape=outs,
    )(xc, xd, deg[0], deg[1], deg[2], deg[3], deg[4], deg[5], w1i, w1t, w1a)


def _leaky(x):
    return jnp.where(x >= 0, x, 0.01 * x)


def _dense2_body(a1i, a1t, a1a, ndi, ndt, nda, nsi, nst,
                 b1i, b1t, b1a, w2i, w2t, g2i, g2t):
    h_chem = _leaky(ndi[...] * (a1i[0] + a1i[1]) + b1i[...] +
                    ndt[...] * (a1t[0] + a1t[1]) + b1t[...])
    h_dis = _leaky(nda[...] * (a1a[0] + a1a[1]) + b1a[...])
    g2i[...] = jnp.dot(h_chem * nsi[...], w2i[...],
                       preferred_element_type=jnp.float32)
    g2t[...] = jnp.dot(h_dis * nst[...], w2t[...],
                       preferred_element_type=jnp.float32)


def _dense2(a1i, a1t, a1a, ndi, ndt, nda, nsi, nst, b1i, b1t, b1a, w2i, w2t):
    grid = NP // RT
    part = pl.BlockSpec((NC, RT, D), lambda i: (0, i, 0))
    row = pl.BlockSpec((RT, D), lambda i: (i, 0))
    vec = pl.BlockSpec((RT, 1), lambda i: (i, 0))
    bias = pl.BlockSpec((1, D), lambda i: (0, 0))
    wsp = pl.BlockSpec((D, D), lambda i: (0, 0))
    outs = (
        jax.ShapeDtypeStruct((NP, D), jnp.float32),
        jax.ShapeDtypeStruct((NP, D), jnp.float32),
    )
    return pl.pallas_call(
        _dense2_body,
        grid=(grid,),
        in_specs=[part, part, part, vec, vec, vec, vec, vec,
                  bias, bias, bias, wsp, wsp],
        out_specs=(row, row),
        out_shape=outs,
    )(a1i, a1t, a1a, ndi, ndt, nda, nsi, nst, b1i, b1t, b1a, w2i, w2t)


def _dense3_body(a2i, a2t, ndi, ndt, b2i, b2t, out):
    out[...] = (ndi[...] * (a2i[0] + a2i[1]) + b2i[...] +
                ndt[...] * (a2t[0] + a2t[1]) + b2t[...])


def _dense3(a2i, a2t, ndi, ndt, b2i, b2t):
    grid = NP // RT
    part = pl.BlockSpec((NC, RT, D), lambda i: (0, i, 0))
    row = pl.BlockSpec((RT, D), lambda i: (i, 0))
    vec = pl.BlockSpec((RT, 1), lambda i: (i, 0))
    bias = pl.BlockSpec((1, D), lambda i: (0, 0))
    return pl.pallas_call(
        _dense3_body,
        grid=(grid,),
        in_specs=[part, part, vec, vec, bias, bias],
        out_specs=row,
        out_shape=jax.ShapeDtypeStruct((NP, D), jnp.float32),
    )(a2i, a2t, ndi, ndt, b2i, b2t)


def _prep_idx(e):
    pad = N + (jnp.arange(EP - E, dtype=jnp.int32) % (NP - N))
    return jnp.concatenate([e, pad]).reshape(NC * NS * CH, L)


def kernel(x_chemical, x_disease, edge_interacts, edge_affects,
           edge_treated_by,
           W1_interacts, b1_interacts, W1_affects, b1_affects,
           W1_treated_by, b1_treated_by,
           W2_interacts, b2_interacts, W2_affects, b2_affects,
           W2_treated_by, b2_treated_by):
    si = _prep_idx(edge_interacts[0])
    di = _prep_idx(edge_interacts[1])
    sa = _prep_idx(edge_affects[0])
    da = _prep_idx(edge_affects[1])
    st = _prep_idx(edge_treated_by[0])
    dt = _prep_idx(edge_treated_by[1])
    xc = jnp.pad(x_chemical, ((0, NP - N), (0, 0)))
    xd = jnp.pad(x_disease, ((0, NP - N), (0, 0)))

    counts = _degree_kernel((si, di, sa, da, st, dt))
    deg = [counts[k * NP:(k + 1) * NP, None] for k in range(6)]

    g1i, g1t, g1a, ndi, ndt, nda, nsi, nst = _dense1(
        xc, xd, deg, W1_interacts, W1_treated_by, W1_affects)

    a1i, a1t, a1a = _agg_kernel((si, st, sa), (di, dt, da), (g1i, g1t, g1a))
    a1i, a1t, a1a = (a.reshape(NC, NP, D) for a in (a1i, a1t, a1a))

    g2i, g2t = _dense2(a1i, a1t, a1a, ndi, ndt, nda, nsi, nst,
                       b1_interacts[None, :], b1_treated_by[None, :],
                       b1_affects[None, :], W2_interacts, W2_treated_by)

    a2i, a2t = _agg_kernel((si, st), (di, dt), (g2i, g2t))
    a2i, a2t = (a.reshape(NC, NP, D) for a in (a2i, a2t))

    out = _dense3(a2i, a2t, ndi, ndt,
                  b2_interacts[None, :], b2_treated_by[None, :])
    return out[:N]


# speedup vs baseline: 11.5395x; 1.1330x over previous
import functools

import jax
import jax.numpy as jnp
from jax import lax
from jax.experimental import pallas as pl
from jax.experimental.pallas import tpu as pltpu
from jax.experimental.pallas import tpu_sc as plsc

N = 10000
NP = 10240
D = 128
E = 160000
EP = 163840
NC = 2
NS = 16
L = 128
CH = EP // (NC * NS * L)
RT = NP // NS
BR = 1024


def _deg_body(a0, a1, a2, a3, a4, a5, out_hbm,
              cidx_v, ones_v, z_v, c0, c1, c2, sem):
    c = lax.axis_index("c")
    s = lax.axis_index("s")
    cnts = (c0, c1, c2)
    for i in range(8):
        ones_v[pl.ds(i * 16, 16)] = jnp.full((16,), 1.0, jnp.float32)

    def _zrow(i, _):
        z_v[pl.ds(i * 16, 16)] = jnp.zeros((16,), jnp.float32)
        return 0

    lax.fori_loop(0, RT // 16, _zrow, 0)
    for k in range(3):
        pltpu.sync_copy(z_v, cnts[k].at[pl.ds(s * RT, RT)])
    plsc.subcore_barrier()
    for cc, arrs in ((0, (a0, a1, a2)), (1, (a3, a4, a5))):
        @pl.when(c == cc)
        def _():
            for k in range(3):
                for h in range(NC):
                    base = (h * NS + s) * CH
                    pltpu.sync_copy(arrs[k].at[pl.ds(base, CH)], cidx_v)

                    def _batch(b, _, k=k):
                        for q in range(8):
                            pltpu.async_copy(
                                ones_v, cnts[k].at[cidx_v.at[8 * b + q]],
                                sem, add=True)
                        for q in range(8):
                            pltpu.make_async_copy(
                                ones_v, cnts[k].at[cidx_v.at[8 * b + q]],
                                sem).wait()
                        return 0

                    lax.fori_loop(0, CH // 8, _batch, 0)
    plsc.subcore_barrier()
    for k in range(3):
        pltpu.sync_copy(cnts[k].at[pl.ds(s * RT, RT)],
                        out_hbm.at[pl.ds((3 * c + k) * NP + s * RT, RT)])


def _degree_kernel(arrs):
    mesh = plsc.VectorSubcoreMesh(core_axis_name="c", subcore_axis_name="s")
    return pl.kernel(
        _deg_body,
        out_type=jax.ShapeDtypeStruct((6 * NP,), jnp.float32),
        mesh=mesh,
        scratch_types=[
            pltpu.VMEM((CH, L), jnp.int32),
            pltpu.VMEM((L,), jnp.float32),
            pltpu.VMEM((RT,), jnp.float32),
            pltpu.VMEM_SHARED((NP,), jnp.float32),
            pltpu.VMEM_SHARED((NP,), jnp.float32),
            pltpu.VMEM_SHARED((NP,), jnp.float32),
            pltpu.SemaphoreType.DMA,
        ],
    )(*arrs)


def _agg_body(n_rel, *args):
    srcs = args[0:n_rel]
    dsts = args[n_rel:2 * n_rel]
    tabs = args[2 * n_rel:3 * n_rel]
    outs = args[3 * n_rel:4 * n_rel]
    sidx_v, didx_v, rows_a, rows_b, acc_sh, sem_a, sem_b = \
        args[4 * n_rel:]
    c = lax.axis_index("c")
    s = lax.axis_index("s")

    def _zrow(i, _):
        for q in range(8):
            rows_a[i, pl.ds(q * 16, 16)] = jnp.zeros((16,), jnp.float32)
        return 0

    for r in range(n_rel):
        lax.fori_loop(0, L, _zrow, 0)
        for kk in range(RT // L):
            pltpu.async_copy(rows_a, acc_sh.at[pl.ds(s * RT + kk * L, L)],
                             sem_a)
        for kk in range(RT // L):
            pltpu.make_async_copy(rows_a,
                                  acc_sh.at[pl.ds(s * RT + kk * L, L)],
                                  sem_a).wait()
        plsc.subcore_barrier()
        pltpu.sync_copy(srcs[r].at[pl.ds((c * NS + s) * CH, CH)], sidx_v)
        pltpu.sync_copy(dsts[r].at[pl.ds((c * NS + s) * CH, CH)], didx_v)

        pltpu.async_copy(tabs[r].at[sidx_v.at[0]], rows_a, sem_a)

        def _chunk2(i, _, r=r):
            j = 2 * i
            pltpu.async_copy(tabs[r].at[sidx_v.at[j + 1]], rows_b, sem_b)
            pltpu.make_async_copy(tabs[r].at[sidx_v.at[0]], rows_a,
                                  sem_a).wait()
            pltpu.sync_copy(rows_a, acc_sh.at[didx_v.at[j]], add=True)

            @pl.when(j + 2 < CH)
            def _():
                pltpu.async_copy(tabs[r].at[sidx_v.at[j + 2]], rows_a, sem_a)

            pltpu.make_async_copy(tabs[r].at[sidx_v.at[0]], rows_b,
                                  sem_b).wait()
            pltpu.sync_copy(rows_b, acc_sh.at[didx_v.at[j + 1]], add=True)
            return 0

        lax.fori_loop(0, CH // 2, _chunk2, 0)
        plsc.subcore_barrier()
        pltpu.sync_copy(acc_sh.at[pl.ds(s * RT, RT)],
                        outs[r].at[pl.ds(c * NP + s * RT, RT)])


def _agg_kernel(srcs, dsts, tabs):
    n_rel = len(srcs)
    mesh = plsc.VectorSubcoreMesh(core_axis_name="c", subcore_axis_name="s")
    out_t = tuple(jax.ShapeDtypeStruct((NC * NP, D), jnp.float32)
                  for _ in range(n_rel))
    return pl.kernel(
        functools.partial(_agg_body, n_rel),
        out_type=out_t,
        mesh=mesh,
        scratch_types=[
            pltpu.VMEM((CH, L), jnp.int32),
            pltpu.VMEM((CH, L), jnp.int32),
            pltpu.VMEM((L, D), jnp.float32),
            pltpu.VMEM((L, D), jnp.float32),
            pltpu.VMEM_SHARED((NP, D), jnp.float32),
            pltpu.SemaphoreType.DMA,
            pltpu.SemaphoreType.DMA,
        ],
    )(*srcs, *dsts, *tabs)


def _norm(deg):
    return jnp.where(deg > 0, lax.rsqrt(jnp.maximum(deg, 1e-12)), 0.0)


def _dense1_body(xc, xd, odi, idi, oda, ida, odt, idt, wi, wt, wa,
                 g1i, g1t, g1a, ndi, ndt, nda, nsi, nst):
    nsi_v = _norm(odi[...])
    nst_v = _norm(odt[...])
    nsa_v = _norm(oda[...])
    ndi[...] = _norm(idi[...])
    nda[...] = _norm(ida[...])
    ndt[...] = _norm(idt[...])
    nsi[...] = nsi_v
    nst[...] = nst_v
    xc_v = xc[...]
    xd_v = xd[...]
    g1i[...] = jnp.dot(xc_v * nsi_v[:, None], wi[...],
                       preferred_element_type=jnp.float32)
    g1t[...] = jnp.dot(xd_v * nst_v[:, None], wt[...],
                       preferred_element_type=jnp.float32)
    g1a[...] = jnp.dot(xc_v * nsa_v[:, None], wa[...],
                       preferred_element_type=jnp.float32)


def _dense1(xc, xd, counts, w1i, w1t, w1a):
    grid = NP // BR
    row = pl.BlockSpec((BR, D), lambda i: (i, 0))
    wsp = pl.BlockSpec((D, D), lambda i: (0, 0))
    nb = NP // BR

    def _cnt(k):
        return pl.BlockSpec((BR,), lambda i, k=k: (k * nb + i,))

    vec = pl.BlockSpec((BR,), lambda i: (i,))
    outs = (
        jax.ShapeDtypeStruct((NP, D), jnp.float32),
        jax.ShapeDtypeStruct((NP, D), jnp.float32),
        jax.ShapeDtypeStruct((NP, D), jnp.float32),
        jax.ShapeDtypeStruct((NP,), jnp.float32),
        jax.ShapeDtypeStruct((NP,), jnp.float32),
        jax.ShapeDtypeStruct((NP,), jnp.float32),
        jax.ShapeDtypeStruct((NP,), jnp.float32),
        jax.ShapeDtypeStruct((NP,), jnp.float32),
    )
    return pl.pallas_call(
        _dense1_body,
        grid=(grid,),
        in_specs=[row, row, _cnt(0), _cnt(1), _cnt(2), _cnt(3), _cnt(4),
                  _cnt(5), wsp, wsp, wsp],
        out_specs=(row, row, row, vec, vec, vec, vec, vec),
        out_shape=outs,
    )(xc, xd, counts, counts, counts, counts, counts, counts,
      w1i, w1t, w1a)


def _leaky(x):
    return jnp.where(x >= 0, x, 0.01 * x)


def _dense2_body(a1i, a1t, a1a, ndi, ndt, nda, nsi, nst,
                 b1i, b1t, b1a, w2i, w2t, g2i, g2t):
    h_chem = _leaky(ndi[...][:, None] * (a1i[0] + a1i[1]) + b1i[...] +
                    ndt[...][:, None] * (a1t[0] + a1t[1]) + b1t[...])
    h_dis = _leaky(nda[...][:, None] * (a1a[0] + a1a[1]) + b1a[...])
    g2i[...] = jnp.dot(h_chem * nsi[...][:, None], w2i[...],
                       preferred_element_type=jnp.float32)
    g2t[...] = jnp.dot(h_dis * nst[...][:, None], w2t[...],
                       preferred_element_type=jnp.float32)


def _dense2(a1i, a1t, a1a, ndi, ndt, nda, nsi, nst, b1i, b1t, b1a, w2i, w2t):
    grid = NP // BR
    part = pl.BlockSpec((NC, BR, D), lambda i: (0, i, 0))
    row = pl.BlockSpec((BR, D), lambda i: (i, 0))
    vec = pl.BlockSpec((BR,), lambda i: (i,))
    bias = pl.BlockSpec((1, D), lambda i: (0, 0))
    wsp = pl.BlockSpec((D, D), lambda i: (0, 0))
    outs = (
        jax.ShapeDtypeStruct((NP, D), jnp.float32),
        jax.ShapeDtypeStruct((NP, D), jnp.float32),
    )
    return pl.pallas_call(
        _dense2_body,
        grid=(grid,),
        in_specs=[part, part, part, vec, vec, vec, vec, vec,
                  bias, bias, bias, wsp, wsp],
        out_specs=(row, row),
        out_shape=outs,
    )(a1i, a1t, a1a, ndi, ndt, nda, nsi, nst, b1i, b1t, b1a, w2i, w2t)


def _dense3_body(a2i, a2t, ndi, ndt, b2i, b2t, out):
    out[...] = (ndi[...][:, None] * (a2i[0] + a2i[1]) + b2i[...] +
                ndt[...][:, None] * (a2t[0] + a2t[1]) + b2t[...])


def _dense3(a2i, a2t, ndi, ndt, b2i, b2t):
    grid = NP // BR
    part = pl.BlockSpec((NC, BR, D), lambda i: (0, i, 0))
    row = pl.BlockSpec((BR, D), lambda i: (i, 0))
    vec = pl.BlockSpec((BR,), lambda i: (i,))
    bias = pl.BlockSpec((1, D), lambda i: (0, 0))
    return pl.pallas_call(
        _dense3_body,
        grid=(grid,),
        in_specs=[part, part, vec, vec, bias, bias],
        out_specs=row,
        out_shape=jax.ShapeDtypeStruct((N, D), jnp.float32),
    )(a2i, a2t, ndi, ndt, b2i, b2t)


def _prep_idx(e):
    pad = N + (jnp.arange(EP - E, dtype=jnp.int32) % (NP - N))
    return jnp.concatenate([e, pad]).reshape(NC * NS * CH, L)


def kernel(x_chemical, x_disease, edge_interacts, edge_affects,
           edge_treated_by,
           W1_interacts, b1_interacts, W1_affects, b1_affects,
           W1_treated_by, b1_treated_by,
           W2_interacts, b2_interacts, W2_affects, b2_affects,
           W2_treated_by, b2_treated_by):
    si = _prep_idx(edge_interacts[0])
    di = _prep_idx(edge_interacts[1])
    sa = _prep_idx(edge_affects[0])
    da = _prep_idx(edge_affects[1])
    st = _prep_idx(edge_treated_by[0])
    dt = _prep_idx(edge_treated_by[1])
    xc = jnp.pad(x_chemical, ((0, NP - N), (0, 0)))
    xd = jnp.pad(x_disease, ((0, NP - N), (0, 0)))

    counts = _degree_kernel((si, di, sa, da, st, dt))

    g1i, g1t, g1a, ndi, ndt, nda, nsi, nst = _dense1(
        xc, xd, counts, W1_interacts, W1_treated_by, W1_affects)

    a1i, a1t, a1a = _agg_kernel((si, st, sa), (di, dt, da), (g1i, g1t, g1a))
    a1i, a1t, a1a = (a.reshape(NC, NP, D) for a in (a1i, a1t, a1a))

    g2i, g2t = _dense2(a1i, a1t, a1a, ndi, ndt, nda, nsi, nst,
                       b1_interacts[None, :], b1_treated_by[None, :],
                       b1_affects[None, :], W2_interacts, W2_treated_by)

    a2i, a2t = _agg_kernel((si, st), (di, dt), (g2i, g2t))
    a2i, a2t = (a.reshape(NC, NP, D) for a in (a2i, a2t))

    return _dense3(a2i, a2t, ndi, ndt,
                   b2_interacts[None, :], b2_treated_by[None, :])


# speedup vs baseline: 11.5459x; 1.0005x over previous
import functools

import jax
import jax.numpy as jnp
from jax import lax
from jax.experimental import pallas as pl
from jax.experimental.pallas import tpu as pltpu
from jax.experimental.pallas import tpu_sc as plsc

N = 10000
NP = 10240
D = 128
E = 160000
EP = 163840
NC = 2
NS = 16
L = 128
CH = EP // (NC * NS * L)
RT = NP // NS
BR = 1024


def _deg_body(a0, a1, a2, a3, a4, a5, out_hbm,
              cidx_v, ones_v, z_v, c0, c1, c2, sem):
    c = lax.axis_index("c")
    s = lax.axis_index("s")
    cnts = (c0, c1, c2)
    for i in range(8):
        ones_v[pl.ds(i * 16, 16)] = jnp.full((16,), 1.0, jnp.float32)

    def _zrow(i, _):
        z_v[pl.ds(i * 16, 16)] = jnp.zeros((16,), jnp.float32)
        return 0

    lax.fori_loop(0, RT // 16, _zrow, 0)
    for k in range(3):
        pltpu.sync_copy(z_v, cnts[k].at[pl.ds(s * RT, RT)])
    plsc.subcore_barrier()
    for cc, arrs in ((0, (a0, a1, a2)), (1, (a3, a4, a5))):
        @pl.when(c == cc)
        def _():
            for k in range(3):
                for h in range(NC):
                    base = (h * NS + s) * CH
                    pltpu.sync_copy(arrs[k].at[pl.ds(base, CH)], cidx_v)

                    def _batch(b, _, k=k):
                        for q in range(8):
                            pltpu.async_copy(
                                ones_v, cnts[k].at[cidx_v.at[8 * b + q]],
                                sem, add=True)
                        for q in range(8):
                            pltpu.make_async_copy(
                                ones_v, cnts[k].at[cidx_v.at[8 * b + q]],
                                sem).wait()
                        return 0

                    lax.fori_loop(0, CH // 8, _batch, 0)
    plsc.subcore_barrier()
    for k in range(3):
        pltpu.sync_copy(cnts[k].at[pl.ds(s * RT, RT)],
                        out_hbm.at[pl.ds((3 * c + k) * NP + s * RT, RT)])


def _degree_kernel(arrs):
    mesh = plsc.VectorSubcoreMesh(core_axis_name="c", subcore_axis_name="s")
    return pl.kernel(
        _deg_body,
        out_type=jax.ShapeDtypeStruct((6 * NP,), jnp.float32),
        mesh=mesh,
        scratch_types=[
            pltpu.VMEM((CH, L), jnp.int32),
            pltpu.VMEM((L,), jnp.float32),
            pltpu.VMEM((RT,), jnp.float32),
            pltpu.VMEM_SHARED((NP,), jnp.float32),
            pltpu.VMEM_SHARED((NP,), jnp.float32),
            pltpu.VMEM_SHARED((NP,), jnp.float32),
            pltpu.SemaphoreType.DMA,
        ],
    )(*arrs)


def _agg_body(n_rel, *args):
    srcs = args[0:n_rel]
    dsts = args[n_rel:2 * n_rel]
    tabs = args[2 * n_rel:3 * n_rel]
    outs = args[3 * n_rel:4 * n_rel]
    sidx_v, didx_v, rows_a, rows_b, acc_sh, sem_a, sem_b = \
        args[4 * n_rel:]
    c = lax.axis_index("c")
    s = lax.axis_index("s")

    def _zrow(i, _):
        for q in range(8):
            rows_a[i, pl.ds(q * 16, 16)] = jnp.zeros((16,), jnp.float32)
        return 0

    for r in range(n_rel):
        lax.fori_loop(0, L, _zrow, 0)
        for kk in range(RT // L):
            pltpu.async_copy(rows_a, acc_sh.at[pl.ds(s * RT + kk * L, L)],
                             sem_a)
        for kk in range(RT // L):
            pltpu.make_async_copy(rows_a,
                                  acc_sh.at[pl.ds(s * RT + kk * L, L)],
                                  sem_a).wait()
        plsc.subcore_barrier()
        pltpu.sync_copy(srcs[r].at[pl.ds((c * NS + s) * CH, CH)], sidx_v)
        pltpu.sync_copy(dsts[r].at[pl.ds((c * NS + s) * CH, CH)], didx_v)

        pltpu.async_copy(tabs[r].at[sidx_v.at[0]], rows_a, sem_a)

        def _chunk2(i, _, r=r):
            j = 2 * i
            pltpu.async_copy(tabs[r].at[sidx_v.at[j + 1]], rows_b, sem_b)
            pltpu.make_async_copy(tabs[r].at[sidx_v.at[0]], rows_a,
                                  sem_a).wait()
            pltpu.sync_copy(rows_a, acc_sh.at[didx_v.at[j]], add=True)

            @pl.when(j + 2 < CH)
            def _():
                pltpu.async_copy(tabs[r].at[sidx_v.at[j + 2]], rows_a, sem_a)

            pltpu.make_async_copy(tabs[r].at[sidx_v.at[0]], rows_b,
                                  sem_b).wait()
            pltpu.sync_copy(rows_b, acc_sh.at[didx_v.at[j + 1]], add=True)
            return 0

        lax.fori_loop(0, CH // 2, _chunk2, 0)
        plsc.subcore_barrier()
        pltpu.sync_copy(acc_sh.at[pl.ds(s * RT, RT)],
                        outs[r].at[pl.ds(c * NP + s * RT, RT)])


def _agg_kernel(srcs, dsts, tabs):
    n_rel = len(srcs)
    mesh = plsc.VectorSubcoreMesh(core_axis_name="c", subcore_axis_name="s")
    out_t = tuple(jax.ShapeDtypeStruct((NC * NP, D), jnp.float32)
                  for _ in range(n_rel))
    return pl.kernel(
        functools.partial(_agg_body, n_rel),
        out_type=out_t,
        mesh=mesh,
        scratch_types=[
            pltpu.VMEM((CH, L), jnp.int32),
            pltpu.VMEM((CH, L), jnp.int32),
            pltpu.VMEM((L, D), jnp.float32),
            pltpu.VMEM((L, D), jnp.float32),
            pltpu.VMEM_SHARED((NP, D), jnp.float32),
            pltpu.SemaphoreType.DMA,
            pltpu.SemaphoreType.DMA,
        ],
    )(*srcs, *dsts, *tabs)


def _norm(deg):
    return jnp.where(deg > 0, lax.rsqrt(jnp.maximum(deg, 1e-12)), 0.0)


def _dense1_body(xc, xd, odi, idi, oda, ida, odt, idt, wi, wt, wa,
                 g1i, g1t, g1a, ndi, ndt, nda, nsi, nst):
    nsi_v = _norm(odi[...])
    nst_v = _norm(odt[...])
    nsa_v = _norm(oda[...])
    ndi[...] = _norm(idi[...])
    nda[...] = _norm(ida[...])
    ndt[...] = _norm(idt[...])
    nsi[...] = nsi_v
    nst[...] = nst_v
    xc_v = xc[...]
    xd_v = xd[...]
    g1i[...] = jnp.dot(xc_v * nsi_v[:, None], wi[...],
                       preferred_element_type=jnp.float32)
    g1t[...] = jnp.dot(xd_v * nst_v[:, None], wt[...],
                       preferred_element_type=jnp.float32)
    g1a[...] = jnp.dot(xc_v * nsa_v[:, None], wa[...],
                       preferred_element_type=jnp.float32)


def _dense1(xc, xd, counts, w1i, w1t, w1a):
    grid = NP // BR
    row = pl.BlockSpec((BR, D), lambda i: (i, 0))
    wsp = pl.BlockSpec((D, D), lambda i: (0, 0))
    nb = NP // BR

    def _cnt(k):
        return pl.BlockSpec((BR,), lambda i, k=k: (k * nb + i,))

    vec = pl.BlockSpec((BR,), lambda i: (i,))
    outs = (
        jax.ShapeDtypeStruct((NP, D), jnp.float32),
        jax.ShapeDtypeStruct((NP, D), jnp.float32),
        jax.ShapeDtypeStruct((NP, D), jnp.float32),
        jax.ShapeDtypeStruct((NP,), jnp.float32),
        jax.ShapeDtypeStruct((NP,), jnp.float32),
        jax.ShapeDtypeStruct((NP,), jnp.float32),
        jax.ShapeDtypeStruct((NP,), jnp.float32),
        jax.ShapeDtypeStruct((NP,), jnp.float32),
    )
    return pl.pallas_call(
        _dense1_body,
        grid=(grid,),
        in_specs=[row, row, _cnt(0), _cnt(1), _cnt(2), _cnt(3), _cnt(4),
                  _cnt(5), wsp, wsp, wsp],
        out_specs=(row, row, row, vec, vec, vec, vec, vec),
        out_shape=outs,
    )(xc, xd, counts, counts, counts, counts, counts, counts,
      w1i, w1t, w1a)


def _leaky(x):
    return jnp.where(x >= 0, x, 0.01 * x)


def _dense2_body(a1i, a1t, a1a, ndi, ndt, nda, nsi, nst,
                 b1i, b1t, b1a, w2i, w2t, g2i, g2t):
    p1i = a1i[0].astype(jnp.float32) + a1i[1].astype(jnp.float32)
    p1t = a1t[0].astype(jnp.float32) + a1t[1].astype(jnp.float32)
    p1a = a1a[0].astype(jnp.float32) + a1a[1].astype(jnp.float32)
    h_chem = _leaky(ndi[...][:, None] * p1i + b1i[...] +
                    ndt[...][:, None] * p1t + b1t[...])
    h_dis = _leaky(nda[...][:, None] * p1a + b1a[...])
    g2i[...] = jnp.dot(h_chem * nsi[...][:, None], w2i[...],
                       preferred_element_type=jnp.float32)
    g2t[...] = jnp.dot(h_dis * nst[...][:, None], w2t[...],
                       preferred_element_type=jnp.float32)


def _dense2(a1i, a1t, a1a, ndi, ndt, nda, nsi, nst, b1i, b1t, b1a, w2i, w2t):
    grid = NP // BR
    part = pl.BlockSpec((NC, BR, D), lambda i: (0, i, 0))
    row = pl.BlockSpec((BR, D), lambda i: (i, 0))
    vec = pl.BlockSpec((BR,), lambda i: (i,))
    bias = pl.BlockSpec((1, D), lambda i: (0, 0))
    wsp = pl.BlockSpec((D, D), lambda i: (0, 0))
    outs = (
        jax.ShapeDtypeStruct((NP, D), jnp.float32),
        jax.ShapeDtypeStruct((NP, D), jnp.float32),
    )
    return pl.pallas_call(
        _dense2_body,
        grid=(grid,),
        in_specs=[part, part, part, vec, vec, vec, vec, vec,
                  bias, bias, bias, wsp, wsp],
        out_specs=(row, row),
        out_shape=outs,
    )(a1i, a1t, a1a, ndi, ndt, nda, nsi, nst, b1i, b1t, b1a, w2i, w2t)


def _dense3_body(a2i, a2t, ndi, ndt, b2i, b2t, out):
    p2i = a2i[0].astype(jnp.float32) + a2i[1].astype(jnp.float32)
    p2t = a2t[0].astype(jnp.float32) + a2t[1].astype(jnp.float32)
    out[...] = (ndi[...][:, None] * p2i + b2i[...] +
                ndt[...][:, None] * p2t + b2t[...])


def _dense3(a2i, a2t, ndi, ndt, b2i, b2t):
    grid = NP // BR
    part = pl.BlockSpec((NC, BR, D), lambda i: (0, i, 0))
    row = pl.BlockSpec((BR, D), lambda i: (i, 0))
    vec = pl.BlockSpec((BR,), lambda i: (i,))
    bias = pl.BlockSpec((1, D), lambda i: (0, 0))
    return pl.pallas_call(
        _dense3_body,
        grid=(grid,),
        in_specs=[part, part, vec, vec, bias, bias],
        out_specs=row,
        out_shape=jax.ShapeDtypeStruct((N, D), jnp.float32),
    )(a2i, a2t, ndi, ndt, b2i, b2t)


def _prep_idx(e):
    pad = N + (jnp.arange(EP - E, dtype=jnp.int32) % (NP - N))
    return jnp.concatenate([e, pad]).reshape(NC * NS * CH, L)


def kernel(x_chemical, x_disease, edge_interacts, edge_affects,
           edge_treated_by,
           W1_interacts, b1_interacts, W1_affects, b1_affects,
           W1_treated_by, b1_treated_by,
           W2_interacts, b2_interacts, W2_affects, b2_affects,
           W2_treated_by, b2_treated_by):
    si = _prep_idx(edge_interacts[0])
    di = _prep_idx(edge_interacts[1])
    sa = _prep_idx(edge_affects[0])
    da = _prep_idx(edge_affects[1])
    st = _prep_idx(edge_treated_by[0])
    dt = _prep_idx(edge_treated_by[1])
    xc = jnp.pad(x_chemical, ((0, NP - N), (0, 0)))
    xd = jnp.pad(x_disease, ((0, NP - N), (0, 0)))

    counts = _degree_kernel((si, di, sa, da, st, dt))

    g1i, g1t, g1a, ndi, ndt, nda, nsi, nst = _dense1(
        xc, xd, counts, W1_interacts, W1_treated_by, W1_affects)

    a1i, a1t, a1a = _agg_kernel((si, st, sa), (di, dt, da), (g1i, g1t, g1a))
    a1i, a1t, a1a = (a.reshape(NC, NP, D) for a in (a1i, a1t, a1a))

    g2i, g2t = _dense2(a1i, a1t, a1a, ndi, ndt, nda, nsi, nst,
                       b1_interacts[None, :], b1_treated_by[None, :],
                       b1_affects[None, :], W2_interacts, W2_treated_by)

    a2i, a2t = _agg_kernel((si, st), (di, dt), (g2i, g2t))
    a2i, a2t = (a.reshape(NC, NP, D) for a in (a2i, a2t))

    return _dense3(a2i, a2t, ndi, ndt,
                   b2_interacts[None, :], b2_treated_by[None, :])


# speedup vs baseline: 11.7491x; 1.0176x over previous
import functools

import jax
import jax.numpy as jnp
from jax import lax
from jax.experimental import pallas as pl
from jax.experimental.pallas import tpu as pltpu
from jax.experimental.pallas import tpu_sc as plsc

N = 10000
NP = 10240
D = 128
E = 160000
EP = 163840
NC = 2
NS = 16
L = 128
CH = EP // (NC * NS * L)
RT = NP // NS
BR = 1024


def _deg_body(a0, a1, a2, a3, a4, a5, out_hbm,
              cidx_v, ones_v, z_v, c0, c1, c2, sem):
    c = lax.axis_index("c")
    s = lax.axis_index("s")
    cnts = (c0, c1, c2)
    for i in range(8):
        ones_v[pl.ds(i * 16, 16)] = jnp.full((16,), 1.0, jnp.float32)

    def _zrow(i, _):
        z_v[pl.ds(i * 16, 16)] = jnp.zeros((16,), jnp.float32)
        return 0

    lax.fori_loop(0, RT // 16, _zrow, 0)
    for k in range(3):
        pltpu.sync_copy(z_v, cnts[k].at[pl.ds(s * RT, RT)])
    plsc.subcore_barrier()
    for cc, arrs in ((0, (a0, a1, a2)), (1, (a3, a4, a5))):
        @pl.when(c == cc)
        def _():
            for k in range(3):
                for h in range(NC):
                    base = (h * NS + s) * CH
                    pltpu.sync_copy(arrs[k].at[pl.ds(base, CH)], cidx_v)

                    def _batch(b, _, k=k):
                        for q in range(10):
                            pltpu.async_copy(
                                ones_v, cnts[k].at[cidx_v.at[10 * b + q]],
                                sem, add=True)
                        for q in range(10):
                            pltpu.make_async_copy(
                                ones_v, cnts[k].at[cidx_v.at[10 * b + q]],
                                sem).wait()
                        return 0

                    lax.fori_loop(0, CH // 10, _batch, 0)
    plsc.subcore_barrier()
    for k in range(3):
        pltpu.sync_copy(cnts[k].at[pl.ds(s * RT, RT)],
                        out_hbm.at[pl.ds((3 * c + k) * NP + s * RT, RT)])


def _degree_kernel(arrs):
    mesh = plsc.VectorSubcoreMesh(core_axis_name="c", subcore_axis_name="s")
    return pl.kernel(
        _deg_body,
        out_type=jax.ShapeDtypeStruct((6 * NP,), jnp.float32),
        mesh=mesh,
        scratch_types=[
            pltpu.VMEM((CH, L), jnp.int32),
            pltpu.VMEM((L,), jnp.float32),
            pltpu.VMEM((RT,), jnp.float32),
            pltpu.VMEM_SHARED((NP,), jnp.float32),
            pltpu.VMEM_SHARED((NP,), jnp.float32),
            pltpu.VMEM_SHARED((NP,), jnp.float32),
            pltpu.SemaphoreType.DMA,
        ],
    )(*arrs)


def _agg_body(n_rel, *args):
    srcs = args[0:n_rel]
    dsts = args[n_rel:2 * n_rel]
    tabs = args[2 * n_rel:3 * n_rel]
    outs = args[3 * n_rel:4 * n_rel]
    sidx_v, didx_v, rows_a, rows_b, acc_sh, sem_a, sem_b = \
        args[4 * n_rel:]
    c = lax.axis_index("c")
    s = lax.axis_index("s")

    def _zrow(i, _):
        for q in range(8):
            rows_a[i, pl.ds(q * 16, 16)] = jnp.zeros((16,), jnp.float32)
        return 0

    for r in range(n_rel):
        pltpu.async_copy(srcs[r].at[pl.ds((c * NS + s) * CH, CH)], sidx_v,
                         sem_b)
        pltpu.async_copy(dsts[r].at[pl.ds((c * NS + s) * CH, CH)], didx_v,
                         sem_b)
        lax.fori_loop(0, L, _zrow, 0)
        for kk in range(RT // L):
            pltpu.async_copy(rows_a, acc_sh.at[pl.ds(s * RT + kk * L, L)],
                             sem_a)
        for kk in range(RT // L):
            pltpu.make_async_copy(rows_a,
                                  acc_sh.at[pl.ds(s * RT + kk * L, L)],
                                  sem_a).wait()
        pltpu.make_async_copy(srcs[r].at[pl.ds((c * NS + s) * CH, CH)],
                              sidx_v, sem_b).wait()
        pltpu.make_async_copy(dsts[r].at[pl.ds((c * NS + s) * CH, CH)],
                              didx_v, sem_b).wait()
        plsc.subcore_barrier()

        pltpu.async_copy(tabs[r].at[sidx_v.at[0]], rows_a, sem_a)

        def _chunk2(i, _, r=r):
            j = 2 * i
            pltpu.async_copy(tabs[r].at[sidx_v.at[j + 1]], rows_b, sem_b)
            pltpu.make_async_copy(tabs[r].at[sidx_v.at[0]], rows_a,
                                  sem_a).wait()
            pltpu.sync_copy(rows_a, acc_sh.at[didx_v.at[j]], add=True)

            @pl.when(j + 2 < CH)
            def _():
                pltpu.async_copy(tabs[r].at[sidx_v.at[j + 2]], rows_a, sem_a)

            pltpu.make_async_copy(tabs[r].at[sidx_v.at[0]], rows_b,
                                  sem_b).wait()
            pltpu.sync_copy(rows_b, acc_sh.at[didx_v.at[j + 1]], add=True)
            return 0

        lax.fori_loop(0, CH // 2, _chunk2, 0)
        plsc.subcore_barrier()
        pltpu.sync_copy(acc_sh.at[pl.ds(s * RT, RT)],
                        outs[r].at[pl.ds(c * NP + s * RT, RT)])


def _agg_kernel(srcs, dsts, tabs):
    n_rel = len(srcs)
    mesh = plsc.VectorSubcoreMesh(core_axis_name="c", subcore_axis_name="s")
    out_t = tuple(jax.ShapeDtypeStruct((NC * NP, D), jnp.float32)
                  for _ in range(n_rel))
    return pl.kernel(
        functools.partial(_agg_body, n_rel),
        out_type=out_t,
        mesh=mesh,
        scratch_types=[
            pltpu.VMEM((CH, L), jnp.int32),
            pltpu.VMEM((CH, L), jnp.int32),
            pltpu.VMEM((L, D), jnp.float32),
            pltpu.VMEM((L, D), jnp.float32),
            pltpu.VMEM_SHARED((NP, D), jnp.float32),
            pltpu.SemaphoreType.DMA,
            pltpu.SemaphoreType.DMA,
        ],
    )(*srcs, *dsts, *tabs)


def _norm(deg):
    return jnp.where(deg > 0, lax.rsqrt(jnp.maximum(deg, 1e-12)), 0.0)


def _dense1_body(xc, xd, odi, idi, oda, ida, odt, idt, wi, wt, wa,
                 g1i, g1t, g1a, ndi, ndt, nda, nsi, nst):
    nsi_v = _norm(odi[...])
    nst_v = _norm(odt[...])
    nsa_v = _norm(oda[...])
    ndi[...] = _norm(idi[...])
    nda[...] = _norm(ida[...])
    ndt[...] = _norm(idt[...])
    nsi[...] = nsi_v
    nst[...] = nst_v
    xc_v = xc[...]
    xd_v = xd[...]
    g1i[...] = jnp.dot(xc_v * nsi_v[:, None], wi[...],
                       preferred_element_type=jnp.float32)
    g1t[...] = jnp.dot(xd_v * nst_v[:, None], wt[...],
                       preferred_element_type=jnp.float32)
    g1a[...] = jnp.dot(xc_v * nsa_v[:, None], wa[...],
                       preferred_element_type=jnp.float32)


def _dense1(xc, xd, counts, w1i, w1t, w1a):
    grid = NP // BR
    row = pl.BlockSpec((BR, D), lambda i: (i, 0))
    wsp = pl.BlockSpec((D, D), lambda i: (0, 0))
    nb = NP // BR

    def _cnt(k):
        return pl.BlockSpec((BR,), lambda i, k=k: (k * nb + i,))

    vec = pl.BlockSpec((BR,), lambda i: (i,))
    outs = (
        jax.ShapeDtypeStruct((NP, D), jnp.float32),
        jax.ShapeDtypeStruct((NP, D), jnp.float32),
        jax.ShapeDtypeStruct((NP, D), jnp.float32),
        jax.ShapeDtypeStruct((NP,), jnp.float32),
        jax.ShapeDtypeStruct((NP,), jnp.float32),
        jax.ShapeDtypeStruct((NP,), jnp.float32),
        jax.ShapeDtypeStruct((NP,), jnp.float32),
        jax.ShapeDtypeStruct((NP,), jnp.float32),
    )
    return pl.pallas_call(
        _dense1_body,
        grid=(grid,),
        in_specs=[row, row, _cnt(0), _cnt(1), _cnt(2), _cnt(3), _cnt(4),
                  _cnt(5), wsp, wsp, wsp],
        out_specs=(row, row, row, vec, vec, vec, vec, vec),
        out_shape=outs,
    )(xc, xd, counts, counts, counts, counts, counts, counts,
      w1i, w1t, w1a)


def _leaky(x):
    return jnp.where(x >= 0, x, 0.01 * x)


def _dense2_body(a1i, a1t, a1a, ndi, ndt, nda, nsi, nst,
                 b1i, b1t, b1a, w2i, w2t, g2i, g2t):
    p1i = a1i[0].astype(jnp.float32) + a1i[1].astype(jnp.float32)
    p1t = a1t[0].astype(jnp.float32) + a1t[1].astype(jnp.float32)
    p1a = a1a[0].astype(jnp.float32) + a1a[1].astype(jnp.float32)
    h_chem = _leaky(ndi[...][:, None] * p1i + b1i[...] +
                    ndt[...][:, None] * p1t + b1t[...])
    h_dis = _leaky(nda[...][:, None] * p1a + b1a[...])
    g2i[...] = jnp.dot(h_chem * nsi[...][:, None], w2i[...],
                       preferred_element_type=jnp.float32)
    g2t[...] = jnp.dot(h_dis * nst[...][:, None], w2t[...],
                       preferred_element_type=jnp.float32)


def _dense2(a1i, a1t, a1a, ndi, ndt, nda, nsi, nst, b1i, b1t, b1a, w2i, w2t):
    grid = NP // BR
    part = pl.BlockSpec((NC, BR, D), lambda i: (0, i, 0))
    row = pl.BlockSpec((BR, D), lambda i: (i, 0))
    vec = pl.BlockSpec((BR,), lambda i: (i,))
    bias = pl.BlockSpec((1, D), lambda i: (0, 0))
    wsp = pl.BlockSpec((D, D), lambda i: (0, 0))
    outs = (
        jax.ShapeDtypeStruct((NP, D), jnp.float32),
        jax.ShapeDtypeStruct((NP, D), jnp.float32),
    )
    return pl.pallas_call(
        _dense2_body,
        grid=(grid,),
        in_specs=[part, part, part, vec, vec, vec, vec, vec,
                  bias, bias, bias, wsp, wsp],
        out_specs=(row, row),
        out_shape=outs,
    )(a1i, a1t, a1a, ndi, ndt, nda, nsi, nst, b1i, b1t, b1a, w2i, w2t)


def _dense3_body(a2i, a2t, ndi, ndt, b2i, b2t, out):
    p2i = a2i[0].astype(jnp.float32) + a2i[1].astype(jnp.float32)
    p2t = a2t[0].astype(jnp.float32) + a2t[1].astype(jnp.float32)
    out[...] = (ndi[...][:, None] * p2i + b2i[...] +
                ndt[...][:, None] * p2t + b2t[...])


def _dense3(a2i, a2t, ndi, ndt, b2i, b2t):
    grid = NP // BR
    part = pl.BlockSpec((NC, BR, D), lambda i: (0, i, 0))
    row = pl.BlockSpec((BR, D), lambda i: (i, 0))
    vec = pl.BlockSpec((BR,), lambda i: (i,))
    bias = pl.BlockSpec((1, D), lambda i: (0, 0))
    return pl.pallas_call(
        _dense3_body,
        grid=(grid,),
        in_specs=[part, part, vec, vec, bias, bias],
        out_specs=row,
        out_shape=jax.ShapeDtypeStruct((N, D), jnp.float32),
    )(a2i, a2t, ndi, ndt, b2i, b2t)


def _prep_idx(e):
    pad = N + (jnp.arange(EP - E, dtype=jnp.int32) % (NP - N))
    return jnp.concatenate([e, pad]).reshape(NC * NS * CH, L)


def kernel(x_chemical, x_disease, edge_interacts, edge_affects,
           edge_treated_by,
           W1_interacts, b1_interacts, W1_affects, b1_affects,
           W1_treated_by, b1_treated_by,
           W2_interacts, b2_interacts, W2_affects, b2_affects,
           W2_treated_by, b2_treated_by):
    si = _prep_idx(edge_interacts[0])
    di = _prep_idx(edge_interacts[1])
    sa = _prep_idx(edge_affects[0])
    da = _prep_idx(edge_affects[1])
    st = _prep_idx(edge_treated_by[0])
    dt = _prep_idx(edge_treated_by[1])
    xc = jnp.pad(x_chemical, ((0, NP - N), (0, 0)))
    xd = jnp.pad(x_disease, ((0, NP - N), (0, 0)))

    counts = _degree_kernel((si, di, sa, da, st, dt))

    g1i, g1t, g1a, ndi, ndt, nda, nsi, nst = _dense1(
        xc, xd, counts, W1_interacts, W1_treated_by, W1_affects)

    a1i, a1t, a1a = _agg_kernel((si, st, sa), (di, dt, da), (g1i, g1t, g1a))
    a1i, a1t, a1a = (a.reshape(NC, NP, D) for a in (a1i, a1t, a1a))

    g2i, g2t = _dense2(a1i, a1t, a1a, ndi, ndt, nda, nsi, nst,
                       b1_interacts[None, :], b1_treated_by[None, :],
                       b1_affects[None, :], W2_interacts, W2_treated_by)

    a2i, a2t = _agg_kernel((si, st), (di, dt), (g2i, g2t))
    a2i, a2t = (a.reshape(NC, NP, D) for a in (a2i, a2t))

    return _dense3(a2i, a2t, ndi, ndt,
                   b2_interacts[None, :], b2_treated_by[None, :])
